# Initial kernel scaffold; baseline (speedup 1.0000x reference)
import jax
import jax.numpy as jnp
from jax.experimental import pallas as pl


def kernel(ids, feats, adj, perm1, perm2, W_x1, b_x1, W_n1, b_n1, W_x2, b_x2, W_n2, b_n2, W_fc, b_fc):
    raise NotImplementedError("write your pallas kernel here")


# speedup vs baseline: 2.4352x; 2.4352x over previous
import functools

import jax
import jax.numpy as jnp
from jax import lax
from jax.experimental import pallas as pl
from jax.experimental.pallas import tpu as pltpu
from jax.experimental.pallas import tpu_sc as plsc

N_NODES = 100000
MAX_DEG = 32
D_IN = 256
N_CLASSES = 40
B = 1024
S1, S2 = 25, 10
H = 128

NC, NS = 2, 16
NW = NC * NS
SPW = B // NW
E1 = SPW * S1
E2 = E1 * S2
CH = 200
LB = H // 16


def _proj_body(f_ref, w_ref, b_ref, px_ref, pn_ref):
    acc = jnp.dot(f_ref[...], w_ref[...], preferred_element_type=jnp.float32)
    acc = acc + b_ref[...]
    px_ref[...] = acc[:, :H]
    pn_ref[...] = acc[:, H:]


def _project(feats, wcat, bcat):
    rows = 1000
    grid = N_NODES // rows
    return pl.pallas_call(
        _proj_body,
        grid=(grid,),
        in_specs=[
            pl.BlockSpec((rows, D_IN), lambda i: (i, 0)),
            pl.BlockSpec((D_IN, 2 * H), lambda i: (0, 0)),
            pl.BlockSpec((1, 2 * H), lambda i: (0, 0)),
        ],
        out_specs=[
            pl.BlockSpec((rows, H), lambda i: (i, 0)),
            pl.BlockSpec((rows, H), lambda i: (i, 0)),
        ],
        out_shape=[
            jax.ShapeDtypeStruct((N_NODES, H), jnp.float32),
            jax.ShapeDtypeStruct((N_NODES, H), jnp.float32),
        ],
    )(feats, wcat, bcat)


def _gather_rows(table_hbm, idx_ref, dst_ref, sem):
    pltpu.async_copy(table_hbm.at[idx_ref], dst_ref, sem).wait()


def _worker_id():
    return lax.axis_index("s") * NC + lax.axis_index("c")


def _sc_body(ids_hbm, adjq_hbm, p1_hbm, p2_hbm, px_hbm, pn_hbm,
             h0x_hbm, h0n_hbm, m1x_hbm, m1n_hbm,
             ids0_v, ids0q_v, p1_v, p2_v, adj0_v, ids1_v, ids1q_v, adjc_v,
             ids2_v, rows_v, h0x_v, h0n_v, m1x_v, m1n_v, sem):
    wid = _worker_id()
    seed_base = wid * SPW

    pltpu.sync_copy(p1_hbm, p1_v)
    pltpu.sync_copy(p2_hbm, p2_v)
    pltpu.sync_copy(ids_hbm.at[pl.ds(seed_base, SPW)], ids0_v)

    def q0(i, c):
        ids0q_v[pl.ds(i * 16, 16)] = ids0_v[pl.ds(i * 16, 16)] // 4
        return c
    lax.fori_loop(0, SPW // 16, q0, 0)

    _gather_rows(adjq_hbm, ids0q_v, adj0_v, sem)

    def b1(i, c):
        mv = i * 16 + lax.iota(jnp.int32, 16)
        e = mv // S1
        nid = plsc.load_gather(ids0_v, [e])
        col = (nid % 4) * MAX_DEG + plsc.load_gather(p1_v, [mv % S1])
        val = plsc.load_gather(adj0_v, [e, col])
        ids1_v[pl.ds(i * 16, 16)] = val
        ids1q_v[pl.ds(i * 16, 16)] = val // 4
        return c
    lax.fori_loop(0, E1 // 16, b1, 0)

    for kc in range(E1 // CH):
        _gather_rows(adjq_hbm, ids1q_v.at[pl.ds(kc * CH, CH)], adjc_v, sem)

        def b2(i, c):
            mg = kc * CH * S2 + i * 16
            mv = mg + lax.iota(jnp.int32, 16)
            ent = mv // S2
            nid = plsc.load_gather(ids1_v, [ent])
            col = (nid % 4) * MAX_DEG + plsc.load_gather(p2_v, [mv % S2])
            ids2_v[pl.ds(mg, 16)] = plsc.load_gather(
                adjc_v, [ent - kc * CH, col])
            return c
        lax.fori_loop(0, CH * S2 // 16, b2, 0)

    _gather_rows(px_hbm, ids0_v, rows_v.at[pl.ds(0, SPW)], sem)

    def pd(r, c):
        for l in range(LB):
            h0x_v[r, pl.ds(l * 16, 16)] = jnp.maximum(
                rows_v[r, pl.ds(l * 16, 16)], 0.0)
        return c
    lax.fori_loop(0, SPW, pd, 0)

    def seed_mean(k, relu_rows, out_ref, scale, relu_out):
        spc = CH // S1

        def sb(s, c):
            def eb(e, accs):
                row = s * S1 + e
                vs = [rows_v[row, pl.ds(l * 16, 16)] for l in range(LB)]
                if relu_rows:
                    vs = [jnp.maximum(v, 0.0) for v in vs]
                return tuple(a + v for a, v in zip(accs, vs))
            accs = lax.fori_loop(
                0, S1, eb, tuple(jnp.zeros((16,), jnp.float32)
                                 for _ in range(LB)))
            srow = k * spc + s
            for l in range(LB):
                v = accs[l] * scale
                if relu_out:
                    v = jnp.maximum(v, 0.0)
                out_ref[srow, pl.ds(l * 16, 16)] = v
            return c
        lax.fori_loop(0, spc, sb, 0)

    for k in range(E1 // CH):
        idx = ids1_v.at[pl.ds(k * CH, CH)]
        _gather_rows(px_hbm, idx, rows_v, sem)
        seed_mean(k, True, m1x_v, 1.0 / S1, False)
        _gather_rows(pn_hbm, idx, rows_v, sem)
        seed_mean(k, False, h0n_v, 1.0 / S1, True)

    def zinit(r, c):
        for l in range(LB):
            m1n_v[r, pl.ds(l * 16, 16)] = jnp.zeros((16,), jnp.float32)
        return c
    lax.fori_loop(0, SPW, zinit, 0)

    epc = CH // S2

    def pc(k, c):
        koff = pl.multiple_of(k * CH, CH)
        _gather_rows(pn_hbm, ids2_v.at[pl.ds(koff, CH)], rows_v, sem)

        def eb(e, c2):
            rowb = e * S2
            eg = k * epc + e
            srow = eg // S1
            for l in range(LB):
                v = rows_v[rowb, pl.ds(l * 16, 16)]
                for r in range(1, S2):
                    v = v + rows_v[rowb + r, pl.ds(l * 16, 16)]
                v = jnp.maximum(v * (1.0 / S2), 0.0) * (1.0 / S1)
                plsc.addupdate(m1n_v.at[srow, pl.ds(l * 16, 16)], v)
            return c2
        lax.fori_loop(0, epc, eb, 0)
        return c
    lax.fori_loop(0, E2 // CH, pc, 0)

    pltpu.sync_copy(h0x_v, h0x_hbm.at[pl.ds(seed_base, SPW)])
    pltpu.sync_copy(h0n_v, h0n_hbm.at[pl.ds(seed_base, SPW)])
    pltpu.sync_copy(m1x_v, m1x_hbm.at[pl.ds(seed_base, SPW)])
    pltpu.sync_copy(m1n_v, m1n_hbm.at[pl.ds(seed_base, SPW)])


def _sc_gather(ids, adj, p1, p2, px, pn):
    mesh = plsc.VectorSubcoreMesh(core_axis_name="c", subcore_axis_name="s")
    f32, i32 = jnp.float32, jnp.int32
    out = jax.ShapeDtypeStruct((B, H), f32)
    k = pl.kernel(
        _sc_body,
        out_type=[out, out, out, out],
        mesh=mesh,
        compiler_params=pltpu.CompilerParams(needs_layout_passes=False),
        scratch_types=[
            pltpu.VMEM((SPW,), i32),
            pltpu.VMEM((SPW,), i32),
            pltpu.VMEM((32,), i32),
            pltpu.VMEM((16,), i32),
            pltpu.VMEM((SPW, 128), i32),
            pltpu.VMEM((E1,), i32),
            pltpu.VMEM((E1,), i32),
            pltpu.VMEM((CH, 128), i32),
            pltpu.VMEM((E2,), i32),
            pltpu.VMEM((CH, H), f32),
            pltpu.VMEM((SPW, H), f32),
            pltpu.VMEM((SPW, H), f32),
            pltpu.VMEM((SPW, H), f32),
            pltpu.VMEM((SPW, H), f32),
            pltpu.SemaphoreType.DMA,
        ],
    )
    return k(ids, adj, p1, p2, px, pn)


def _head_body(h0x, h0n, m1x, m1n, wx2, wn2, wfc, bx2, bn2, bfc, out):
    gx = (jnp.dot(h0x[...], wx2[:H, :], preferred_element_type=jnp.float32)
          + jnp.dot(h0n[...], wx2[H:, :], preferred_element_type=jnp.float32)
          + bx2[...])
    gn = (jnp.dot(m1x[...], wn2[:H, :], preferred_element_type=jnp.float32)
          + jnp.dot(m1n[...], wn2[H:, :], preferred_element_type=jnp.float32)
          + bn2[...])
    nrm = (jnp.sum(gx * gx, axis=1, keepdims=True)
           + jnp.sum(gn * gn, axis=1, keepdims=True))
    s = 1.0 / jnp.maximum(jnp.sqrt(nrm), 1e-12)
    out[...] = (jnp.dot(gx * s, wfc[:H, :], preferred_element_type=jnp.float32)
                + jnp.dot(gn * s, wfc[H:, :],
                          preferred_element_type=jnp.float32)
                + bfc[...])


def _head(h0x, h0n, m1x, m1n, wx2, wn2, wfc, bx2, bn2, bfc):
    return pl.pallas_call(
        _head_body,
        out_shape=jax.ShapeDtypeStruct((B, N_CLASSES), jnp.float32),
    )(h0x, h0n, m1x, m1n, wx2, wn2, wfc, bx2, bn2, bfc)


def kernel(ids, feats, adj, perm1, perm2, W_x1, b_x1, W_n1, b_n1,
           W_x2, b_x2, W_n2, b_n2, W_fc, b_fc):
    wcat = jnp.concatenate([W_x1, W_n1], axis=1)
    bcat = jnp.concatenate([b_x1, b_n1]).reshape(1, 2 * H)
    px, pn = _project(feats, wcat, bcat)
    p1 = jnp.pad(perm1, (0, 32 - S1))
    p2 = jnp.pad(perm2, (0, 16 - S2))
    h0x, h0n, m1x, m1n = _sc_gather(ids, adj.reshape(N_NODES // 4, 4 * MAX_DEG),
                                    p1, p2, px, pn)
    return _head(h0x, h0n, m1x, m1n, W_x2, W_n2, W_fc,
                 b_x2.reshape(1, H), b_n2.reshape(1, H),
                 b_fc.reshape(1, N_CLASSES))


# speedup vs baseline: 2.9849x; 1.2257x over previous
import functools

import jax
import jax.numpy as jnp
from jax import lax
from jax.experimental import pallas as pl
from jax.experimental.pallas import tpu as pltpu
from jax.experimental.pallas import tpu_sc as plsc

N_NODES = 100000
MAX_DEG = 32
D_IN = 256
N_CLASSES = 40
B = 1024
S1, S2 = 25, 10
H = 128

NC, NS = 2, 16
NW = NC * NS
SPW = B // NW
E1 = SPW * S1
E2 = E1 * S2
CH = 200
LB = H // 16


def _proj_body(f_ref, w_ref, b_ref, px_ref, pn_ref):
    acc = jnp.dot(f_ref[...], w_ref[...], preferred_element_type=jnp.float32)
    acc = acc + b_ref[...]
    px_ref[...] = acc[:, :H]
    pn_ref[...] = acc[:, H:]


def _project(feats, wcat, bcat):
    rows = 1000
    grid = N_NODES // rows
    return pl.pallas_call(
        _proj_body,
        grid=(grid,),
        in_specs=[
            pl.BlockSpec((rows, D_IN), lambda i: (i, 0)),
            pl.BlockSpec((D_IN, 2 * H), lambda i: (0, 0)),
            pl.BlockSpec((1, 2 * H), lambda i: (0, 0)),
        ],
        out_specs=[
            pl.BlockSpec((rows, H), lambda i: (i, 0)),
            pl.BlockSpec((rows, H), lambda i: (i, 0)),
        ],
        out_shape=[
            jax.ShapeDtypeStruct((N_NODES, H), jnp.float32),
            jax.ShapeDtypeStruct((N_NODES, H), jnp.float32),
        ],
    )(feats, wcat, bcat)


def _gather_rows(table_hbm, idx_ref, dst_ref, sem):
    pltpu.async_copy(table_hbm.at[idx_ref], dst_ref, sem).wait()


def _issue_gather(table_hbm, idx_ref, dst_ref, sem):
    pltpu.async_copy(table_hbm.at[idx_ref], dst_ref, sem)


def _drain_gather(dummy_hbm, dst_ref, sem):
    pltpu.make_async_copy(dummy_hbm, dst_ref, sem).wait()


def _worker_id():
    return lax.axis_index("s") * NC + lax.axis_index("c")


def _sc_body(ids_hbm, adjq_hbm, p1_hbm, p2_hbm, px_hbm, pn_hbm,
             h0x_hbm, h0n_hbm, m1x_hbm, m1n_hbm,
             ids0_v, ids0q_v, p1_v, p2_v, adj0_v, ids1_v, ids1q_v, adjc_v,
             ids2_v, rows_v, rows2_v, h0x_v, h0n_v, m1x_v, m1n_v, sem, sem2):
    wid = _worker_id()
    seed_base = wid * SPW

    pltpu.sync_copy(p1_hbm, p1_v)
    pltpu.sync_copy(p2_hbm, p2_v)
    pltpu.sync_copy(ids_hbm.at[pl.ds(seed_base, SPW)], ids0_v)

    def q0(i, c):
        ids0q_v[pl.ds(i * 16, 16)] = ids0_v[pl.ds(i * 16, 16)] // 4
        return c
    lax.fori_loop(0, SPW // 16, q0, 0)

    _gather_rows(adjq_hbm, ids0q_v, adj0_v, sem)

    def b1(i, c):
        mv = i * 16 + lax.iota(jnp.int32, 16)
        e = mv // S1
        nid = plsc.load_gather(ids0_v, [e])
        col = (nid % 4) * MAX_DEG + plsc.load_gather(p1_v, [mv % S1])
        val = plsc.load_gather(adj0_v, [e, col])
        ids1_v[pl.ds(i * 16, 16)] = val
        ids1q_v[pl.ds(i * 16, 16)] = val // 4
        return c
    lax.fori_loop(0, E1 // 16, b1, 0)

    for kc in range(E1 // CH):
        _gather_rows(adjq_hbm, ids1q_v.at[pl.ds(kc * CH, CH)], adjc_v, sem)

        def b2(i, c):
            mg = kc * CH * S2 + i * 16
            mv = mg + lax.iota(jnp.int32, 16)
            ent = mv // S2
            nid = plsc.load_gather(ids1_v, [ent])
            col = (nid % 4) * MAX_DEG + plsc.load_gather(p2_v, [mv % S2])
            ids2_v[pl.ds(mg, 16)] = plsc.load_gather(
                adjc_v, [ent - kc * CH, col])
            return c
        lax.fori_loop(0, CH * S2 // 16, b2, 0)

    _gather_rows(px_hbm, ids0_v, rows_v.at[pl.ds(0, SPW)], sem)

    def pd(r, c):
        for l in range(LB):
            h0x_v[r, pl.ds(l * 16, 16)] = jnp.maximum(
                rows_v[r, pl.ds(l * 16, 16)], 0.0)
        return c
    lax.fori_loop(0, SPW, pd, 0)

    def seed_mean(buf, k, relu_rows, out_ref, scale, relu_out):
        spc = CH // S1

        def sb(s, c):
            def eb(e, accs):
                row = s * S1 + e
                vs = [buf[row, pl.ds(l * 16, 16)] for l in range(LB)]
                if relu_rows:
                    vs = [jnp.maximum(v, 0.0) for v in vs]
                return tuple(a + v for a, v in zip(accs, vs))
            accs = lax.fori_loop(
                0, S1, eb, tuple(jnp.zeros((16,), jnp.float32)
                                 for _ in range(LB)))
            srow = k * spc + s
            for l in range(LB):
                v = accs[l] * scale
                if relu_out:
                    v = jnp.maximum(v, 0.0)

                out_ref[srow, pl.ds(l * 16, 16)] = v
            return c
        lax.fori_loop(0, spc, sb, 0)

    bufs = (rows_v, rows2_v)
    sems = (sem, sem2)
    units = []
    for k in range(E1 // CH):
        units.append((px_hbm, k, True, m1x_v, 1.0 / S1, False))
        units.append((pn_hbm, k, False, h0n_v, 1.0 / S1, True))

    def issue(u, b):
        table, k = units[u][0], units[u][1]
        _issue_gather(table, ids1_v.at[pl.ds(k * CH, CH)], bufs[b], sems[b])

    issue(0, 0)
    for u in range(len(units)):
        b = u % 2
        if u + 1 < len(units):
            issue(u + 1, (u + 1) % 2)
        _drain_gather(pn_hbm.at[pl.ds(0, CH)], bufs[b], sems[b])
        _, k, rr, oref, sc, ro = units[u]
        seed_mean(bufs[b], k, rr, oref, sc, ro)

    def zinit(r, c):
        for l in range(LB):
            m1n_v[r, pl.ds(l * 16, 16)] = jnp.zeros((16,), jnp.float32)
        return c
    lax.fori_loop(0, SPW, zinit, 0)

    epc = CH // S2
    nch = E2 // CH

    def pc_start(k, b):
        koff = pl.multiple_of(k * CH, CH)
        _issue_gather(pn_hbm, ids2_v.at[pl.ds(koff, CH)], bufs[b], sems[b])

    def pc_compute(buf, k):
        def eb(e, c2):
            rowb = e * S2
            eg = k * epc + e
            srow = eg // S1
            for l in range(LB):
                v = buf[rowb, pl.ds(l * 16, 16)]
                for r in range(1, S2):
                    v = v + buf[rowb + r, pl.ds(l * 16, 16)]
                v = jnp.maximum(v * (1.0 / S2), 0.0) * (1.0 / S1)
                plsc.addupdate(m1n_v.at[srow, pl.ds(l * 16, 16)], v)
            return c2
        lax.fori_loop(0, epc, eb, 0)

    def drain(b):
        _drain_gather(pn_hbm.at[pl.ds(0, CH)], bufs[b], sems[b])

    pc_start(0, 0)

    def pc_pair(kp, c):
        k0 = pl.multiple_of(kp * 2, 2)
        pc_start(k0 + 1, 1)
        drain(0)
        pc_compute(bufs[0], k0)

        @pl.when(kp < nch // 2 - 1)
        def _():
            pc_start(k0 + 2, 0)
        drain(1)
        pc_compute(bufs[1], k0 + 1)
        return c
    lax.fori_loop(0, nch // 2, pc_pair, 0)

    pltpu.sync_copy(h0x_v, h0x_hbm.at[pl.ds(seed_base, SPW)])
    pltpu.sync_copy(h0n_v, h0n_hbm.at[pl.ds(seed_base, SPW)])
    pltpu.sync_copy(m1x_v, m1x_hbm.at[pl.ds(seed_base, SPW)])
    pltpu.sync_copy(m1n_v, m1n_hbm.at[pl.ds(seed_base, SPW)])


def _sc_gather(ids, adj, p1, p2, px, pn):
    mesh = plsc.VectorSubcoreMesh(core_axis_name="c", subcore_axis_name="s")
    f32, i32 = jnp.float32, jnp.int32
    out = jax.ShapeDtypeStruct((B, H), f32)
    k = pl.kernel(
        _sc_body,
        out_type=[out, out, out, out],
        mesh=mesh,
        compiler_params=pltpu.CompilerParams(needs_layout_passes=False),
        scratch_types=[
            pltpu.VMEM((SPW,), i32),
            pltpu.VMEM((SPW,), i32),
            pltpu.VMEM((32,), i32),
            pltpu.VMEM((16,), i32),
            pltpu.VMEM((SPW, 128), i32),
            pltpu.VMEM((E1,), i32),
            pltpu.VMEM((E1,), i32),
            pltpu.VMEM((CH, 128), i32),
            pltpu.VMEM((E2,), i32),
            pltpu.VMEM((CH, H), f32),
            pltpu.VMEM((CH, H), f32),
            pltpu.VMEM((SPW, H), f32),
            pltpu.VMEM((SPW, H), f32),
            pltpu.VMEM((SPW, H), f32),
            pltpu.VMEM((SPW, H), f32),
            pltpu.SemaphoreType.DMA,
            pltpu.SemaphoreType.DMA,
        ],
    )
    return k(ids, adj, p1, p2, px, pn)


def _head_body(h0x, h0n, m1x, m1n, wx2, wn2, wfc, bx2, bn2, bfc, out):
    gx = (jnp.dot(h0x[...], wx2[:H, :], preferred_element_type=jnp.float32)
          + jnp.dot(h0n[...], wx2[H:, :], preferred_element_type=jnp.float32)
          + bx2[...])
    gn = (jnp.dot(m1x[...], wn2[:H, :], preferred_element_type=jnp.float32)
          + jnp.dot(m1n[...], wn2[H:, :], preferred_element_type=jnp.float32)
          + bn2[...])
    nrm = (jnp.sum(gx * gx, axis=1, keepdims=True)
           + jnp.sum(gn * gn, axis=1, keepdims=True))
    s = 1.0 / jnp.maximum(jnp.sqrt(nrm), 1e-12)
    out[...] = (jnp.dot(gx * s, wfc[:H, :], preferred_element_type=jnp.float32)
                + jnp.dot(gn * s, wfc[H:, :],
                          preferred_element_type=jnp.float32)
                + bfc[...])


def _head(h0x, h0n, m1x, m1n, wx2, wn2, wfc, bx2, bn2, bfc):
    return pl.pallas_call(
        _head_body,
        out_shape=jax.ShapeDtypeStruct((B, N_CLASSES), jnp.float32),
    )(h0x, h0n, m1x, m1n, wx2, wn2, wfc, bx2, bn2, bfc)


def kernel(ids, feats, adj, perm1, perm2, W_x1, b_x1, W_n1, b_n1,
           W_x2, b_x2, W_n2, b_n2, W_fc, b_fc):
    wcat = jnp.concatenate([W_x1, W_n1], axis=1)
    bcat = jnp.concatenate([b_x1, b_n1]).reshape(1, 2 * H)
    px, pn = _project(feats, wcat, bcat)
    p1 = jnp.pad(perm1, (0, 32 - S1))
    p2 = jnp.pad(perm2, (0, 16 - S2))
    h0x, h0n, m1x, m1n = _sc_gather(ids, adj.reshape(N_NODES // 4, 4 * MAX_DEG),
                                    p1, p2, px, pn)
    return _head(h0x, h0n, m1x, m1n, W_x2, W_n2, W_fc,
                 b_x2.reshape(1, H), b_n2.reshape(1, H),
                 b_fc.reshape(1, N_CLASSES))


# speedup vs baseline: 3.7251x; 1.2480x over previous
import jax
import jax.numpy as jnp
from jax import lax
from jax.experimental import pallas as pl
from jax.experimental.pallas import tpu as pltpu
from jax.experimental.pallas import tpu_sc as plsc

N_NODES = 100000
MAX_DEG = 32
D_IN = 256
N_CLASSES = 40
B = 1024
S1, S2 = 25, 10
H = 128

NC, NS = 2, 16
NW = NC * NS
SPW = B // NW
E1 = SPW * S1
E2 = E1 * S2
CH = 200
LB = H // 16


def _bf16_bits(x):
    u = jax.lax.bitcast_convert_type(x, jnp.uint32)
    return (u + jnp.uint32(0x7FFF) + ((u >> 16) & jnp.uint32(1))) >> 16


def _proj_body(f_ref, w_ref, b_ref, pxn_ref):
    acc = jnp.dot(f_ref[...], w_ref[...], preferred_element_type=jnp.float32)
    acc = acc + b_ref[...]
    px = _bf16_bits(acc[:, :H])
    pn = _bf16_bits(acc[:, H:])
    pxn_ref[...] = px | (pn << 16)


def _project(feats, wcat, bcat):
    rows = 1000
    grid = N_NODES // rows
    return pl.pallas_call(
        _proj_body,
        grid=(grid,),
        in_specs=[
            pl.BlockSpec((rows, D_IN), lambda i: (i, 0)),
            pl.BlockSpec((D_IN, 2 * H), lambda i: (0, 0)),
            pl.BlockSpec((1, 2 * H), lambda i: (0, 0)),
        ],
        out_specs=pl.BlockSpec((rows, H), lambda i: (i, 0)),
        out_shape=jax.ShapeDtypeStruct((N_NODES, H), jnp.uint32),
    )(feats, wcat, bcat)


def _gather_rows(table_hbm, idx_ref, dst_ref, sem):
    pltpu.async_copy(table_hbm.at[idx_ref], dst_ref, sem).wait()


def _issue_gather(table_hbm, idx_ref, dst_ref, sem):
    pltpu.async_copy(table_hbm.at[idx_ref], dst_ref, sem)


def _drain_gather(dummy_hbm, dst_ref, sem):
    pltpu.make_async_copy(dummy_hbm, dst_ref, sem).wait()


def _worker_id():
    return lax.axis_index("s") * NC + lax.axis_index("c")


def _load_pair(buf, row):
    pxs, pns = [], []
    for blk in range(LB):
        w = buf[row, pl.ds(blk * 16, 16)]
        ab = plsc.bitcast(w, jnp.bfloat16)
        a, b = plsc.unpack(ab, format=plsc.PackFormat.INTERLEAVED)
        pxs.append(a)
        pns.append(b)
    return pxs, pns


def _sc_body(ids_hbm, adjq_hbm, p1_hbm, p2_hbm, pxn_hbm,
             h0x_hbm, h0n_hbm, m1x_hbm, m1n_hbm,
             ids0_v, ids0q_v, p1_v, p2_v, adj0_v, ids1_v, ids1q_v, adjc_v,
             ids2_v, rows_v, rows2_v, h0x_v, h0n_v, m1x_v, m1n_v, sem, sem2):
    wid = _worker_id()
    seed_base = wid * SPW

    pltpu.sync_copy(p1_hbm, p1_v)
    pltpu.sync_copy(p2_hbm, p2_v)
    pltpu.sync_copy(ids_hbm.at[pl.ds(seed_base, SPW)], ids0_v)

    def q0(i, c):
        ids0q_v[pl.ds(i * 16, 16)] = ids0_v[pl.ds(i * 16, 16)] // 4
        return c
    lax.fori_loop(0, SPW // 16, q0, 0)

    _gather_rows(adjq_hbm, ids0q_v, adj0_v, sem)

    def b1(i, c):
        mv = i * 16 + lax.iota(jnp.int32, 16)
        e = mv // S1
        nid = plsc.load_gather(ids0_v, [e])
        col = (nid % 4) * MAX_DEG + plsc.load_gather(p1_v, [mv % S1])
        val = plsc.load_gather(adj0_v, [e, col])
        ids1_v[pl.ds(i * 16, 16)] = val
        ids1q_v[pl.ds(i * 16, 16)] = val // 4
        return c
    lax.fori_loop(0, E1 // 16, b1, 0)

    for kc in range(E1 // CH):
        _gather_rows(adjq_hbm, ids1q_v.at[pl.ds(kc * CH, CH)], adjc_v, sem)

        def b2(i, c):
            mg = kc * CH * S2 + i * 16
            mv = mg + lax.iota(jnp.int32, 16)
            ent = mv // S2
            nid = plsc.load_gather(ids1_v, [ent])
            col = (nid % 4) * MAX_DEG + plsc.load_gather(p2_v, [mv % S2])
            ids2_v[pl.ds(mg, 16)] = plsc.load_gather(
                adjc_v, [ent - kc * CH, col])
            return c
        lax.fori_loop(0, CH * S2 // 16, b2, 0)

    _gather_rows(pxn_hbm, ids0_v, rows_v.at[pl.ds(0, SPW)], sem)

    def pd(r, c):
        pxs, _ = _load_pair(rows_v, r)
        for l in range(LB):
            h0x_v[r, pl.ds(l * 16, 16)] = jnp.maximum(pxs[l], 0.0)
        return c
    lax.fori_loop(0, SPW, pd, 0)

    bufs = (rows_v, rows2_v)
    sems = (sem, sem2)

    def seed_mean2(buf, k):
        spc = CH // S1

        def sb(s, c):
            def eb(e, accs):
                row = s * S1 + e
                pxs, pns = _load_pair(buf, row)
                ax = tuple(a + jnp.maximum(v, 0.0)
                           for a, v in zip(accs[:LB], pxs))
                an = tuple(a + v for a, v in zip(accs[LB:], pns))
                return ax + an
            accs = lax.fori_loop(
                0, S1, eb, tuple(jnp.zeros((16,), jnp.float32)
                                 for _ in range(2 * LB)))
            srow = k * spc + s
            for l in range(LB):
                m1x_v[srow, pl.ds(l * 16, 16)] = accs[l] * (1.0 / S1)
                h0n_v[srow, pl.ds(l * 16, 16)] = jnp.maximum(
                    accs[LB + l] * (1.0 / S1), 0.0)
            return c
        lax.fori_loop(0, spc, sb, 0)

    nab = E1 // CH

    def issue_ab(k, b):
        _issue_gather(pxn_hbm, ids1_v.at[pl.ds(k * CH, CH)], bufs[b], sems[b])

    issue_ab(0, 0)
    for u in range(nab):
        b = u % 2
        if u + 1 < nab:
            issue_ab(u + 1, (u + 1) % 2)
        _drain_gather(pxn_hbm.at[pl.ds(0, CH)], bufs[b], sems[b])
        seed_mean2(bufs[b], u)

    def zinit(r, c):
        for l in range(LB):
            m1n_v[r, pl.ds(l * 16, 16)] = jnp.zeros((16,), jnp.float32)
        return c
    lax.fori_loop(0, SPW, zinit, 0)

    epc = CH // S2
    nch = E2 // CH

    def pc_start(k, b):
        koff = pl.multiple_of(k * CH, CH)
        _issue_gather(pxn_hbm, ids2_v.at[pl.ds(koff, CH)], bufs[b], sems[b])

    def pc_compute(buf, k):
        def eb(e, c2):
            rowb = e * S2
            eg = k * epc + e
            srow = eg // S1
            _, vs = _load_pair(buf, rowb)
            for r in range(1, S2):
                _, vr = _load_pair(buf, rowb + r)
                vs = [a + b2 for a, b2 in zip(vs, vr)]
            for l in range(LB):
                v = jnp.maximum(vs[l] * (1.0 / S2), 0.0) * (1.0 / S1)
                plsc.addupdate(m1n_v.at[srow, pl.ds(l * 16, 16)], v)
            return c2
        lax.fori_loop(0, epc, eb, 0)

    def drain(b):
        _drain_gather(pxn_hbm.at[pl.ds(0, CH)], bufs[b], sems[b])

    pc_start(0, 0)

    def pc_pair(kp, c):
        k0 = pl.multiple_of(kp * 2, 2)
        pc_start(k0 + 1, 1)
        drain(0)
        pc_compute(bufs[0], k0)

        @pl.when(kp < nch // 2 - 1)
        def _():
            pc_start(k0 + 2, 0)
        drain(1)
        pc_compute(bufs[1], k0 + 1)
        return c
    lax.fori_loop(0, nch // 2, pc_pair, 0)

    pltpu.sync_copy(h0x_v, h0x_hbm.at[pl.ds(seed_base, SPW)])
    pltpu.sync_copy(h0n_v, h0n_hbm.at[pl.ds(seed_base, SPW)])
    pltpu.sync_copy(m1x_v, m1x_hbm.at[pl.ds(seed_base, SPW)])
    pltpu.sync_copy(m1n_v, m1n_hbm.at[pl.ds(seed_base, SPW)])


def _sc_gather(ids, adjq, p1, p2, pxn):
    mesh = plsc.VectorSubcoreMesh(core_axis_name="c", subcore_axis_name="s")
    f32, i32 = jnp.float32, jnp.int32
    out = jax.ShapeDtypeStruct((B, H), f32)
    k = pl.kernel(
        _sc_body,
        out_type=[out, out, out, out],
        mesh=mesh,
        compiler_params=pltpu.CompilerParams(needs_layout_passes=False),
        scratch_types=[
            pltpu.VMEM((SPW,), i32),
            pltpu.VMEM((SPW,), i32),
            pltpu.VMEM((32,), i32),
            pltpu.VMEM((16,), i32),
            pltpu.VMEM((SPW, 128), i32),
            pltpu.VMEM((E1,), i32),
            pltpu.VMEM((E1,), i32),
            pltpu.VMEM((CH, 128), i32),
            pltpu.VMEM((E2,), i32),
            pltpu.VMEM((CH, H), jnp.uint32),
            pltpu.VMEM((CH, H), jnp.uint32),
            pltpu.VMEM((SPW, H), f32),
            pltpu.VMEM((SPW, H), f32),
            pltpu.VMEM((SPW, H), f32),
            pltpu.VMEM((SPW, H), f32),
            pltpu.SemaphoreType.DMA,
            pltpu.SemaphoreType.DMA,
        ],
    )
    return k(ids, adjq, p1, p2, pxn)


def _head_body(h0x, h0n, m1x, m1n, wx2, wn2, wfc, bx2, bn2, bfc, out):
    gx = (jnp.dot(h0x[...], wx2[:H, :], preferred_element_type=jnp.float32)
          + jnp.dot(h0n[...], wx2[H:, :], preferred_element_type=jnp.float32)
          + bx2[...])
    gn = (jnp.dot(m1x[...], wn2[:H, :], preferred_element_type=jnp.float32)
          + jnp.dot(m1n[...], wn2[H:, :], preferred_element_type=jnp.float32)
          + bn2[...])
    nrm = (jnp.sum(gx * gx, axis=1, keepdims=True)
           + jnp.sum(gn * gn, axis=1, keepdims=True))
    s = 1.0 / jnp.maximum(jnp.sqrt(nrm), 1e-12)
    out[...] = (jnp.dot(gx * s, wfc[:H, :], preferred_element_type=jnp.float32)
                + jnp.dot(gn * s, wfc[H:, :],
                          preferred_element_type=jnp.float32)
                + bfc[...])


def _head(h0x, h0n, m1x, m1n, wx2, wn2, wfc, bx2, bn2, bfc):
    return pl.pallas_call(
        _head_body,
        out_shape=jax.ShapeDtypeStruct((B, N_CLASSES), jnp.float32),
    )(h0x, h0n, m1x, m1n, wx2, wn2, wfc, bx2, bn2, bfc)


def kernel(ids, feats, adj, perm1, perm2, W_x1, b_x1, W_n1, b_n1,
           W_x2, b_x2, W_n2, b_n2, W_fc, b_fc):
    wcat = jnp.concatenate([W_x1, W_n1], axis=1)
    bcat = jnp.concatenate([b_x1, b_n1]).reshape(1, 2 * H)
    pxn = _project(feats, wcat, bcat)
    p1 = jnp.pad(perm1, (0, 32 - S1))
    p2 = jnp.pad(perm2, (0, 16 - S2))
    h0x, h0n, m1x, m1n = _sc_gather(
        ids, adj.reshape(N_NODES // 4, 4 * MAX_DEG), p1, p2, pxn)
    return _head(h0x, h0n, m1x, m1n, W_x2, W_n2, W_fc,
                 b_x2.reshape(1, H), b_n2.reshape(1, H),
                 b_fc.reshape(1, N_CLASSES))


# speedup vs baseline: 4.2343x; 1.1367x over previous
import jax
import jax.numpy as jnp
from jax import lax
from jax.experimental import pallas as pl
from jax.experimental.pallas import tpu as pltpu
from jax.experimental.pallas import tpu_sc as plsc

N_NODES = 100000
MAX_DEG = 32
D_IN = 256
N_CLASSES = 40
B = 1024
S1, S2 = 25, 10
H = 128

NC, NS = 2, 16
NW = NC * NS
SPW = B // NW
E1 = SPW * S1
E2 = E1 * S2
CH = 200
LB = H // 16


def _bf16_bits(x):
    u = jax.lax.bitcast_convert_type(x, jnp.uint32)
    return (u + jnp.uint32(0x7FFF) + ((u >> 16) & jnp.uint32(1))) >> 16


def _proj_body(f_ref, w_ref, b_ref, pxn_ref):
    acc = jnp.dot(f_ref[...], w_ref[...], preferred_element_type=jnp.float32)
    acc = acc + b_ref[...]
    px = _bf16_bits(acc[:, :H])
    pn = _bf16_bits(acc[:, H:])
    pxn_ref[...] = px | (pn << 16)


def _project(feats, wcat, bcat):
    rows = 2000
    grid = N_NODES // rows
    return pl.pallas_call(
        _proj_body,
        grid=(grid,),
        in_specs=[
            pl.BlockSpec((rows, D_IN), lambda i: (i, 0)),
            pl.BlockSpec((D_IN, 2 * H), lambda i: (0, 0)),
            pl.BlockSpec((1, 2 * H), lambda i: (0, 0)),
        ],
        out_specs=pl.BlockSpec((rows, H), lambda i: (i, 0)),
        out_shape=jax.ShapeDtypeStruct((N_NODES, H), jnp.uint32),
    )(feats, wcat, bcat)


def _gather_rows(table_hbm, idx_ref, dst_ref, sem):
    pltpu.async_copy(table_hbm.at[idx_ref], dst_ref, sem).wait()


def _issue_gather(table_hbm, idx_ref, dst_ref, sem):
    pltpu.async_copy(table_hbm.at[idx_ref], dst_ref, sem)


def _drain_gather(dummy_hbm, dst_ref, sem):
    pltpu.make_async_copy(dummy_hbm, dst_ref, sem).wait()


def _worker_id():
    return lax.axis_index("s") * NC + lax.axis_index("c")


def _load_pair(buf, row):
    pxs, pns = [], []
    for blk in range(LB):
        w = buf[row, pl.ds(blk * 16, 16)]
        ab = plsc.bitcast(w, jnp.bfloat16)
        a, b = plsc.unpack(ab, format=plsc.PackFormat.INTERLEAVED)
        pxs.append(a)
        pns.append(b)
    return pxs, pns


def _ids_body(ids_hbm, adjq_hbm, p1_hbm, p2_hbm,
              ids1_hbm, ids2_hbm,
              ids0_v, ids0q_v, p1_v, p2_v, adj0_v, ids1_v, ids1q_v, adjc_v,
              ids2_v, sem):
    wid = _worker_id()
    seed_base = wid * SPW

    pltpu.sync_copy(p1_hbm, p1_v)
    pltpu.sync_copy(p2_hbm, p2_v)
    pltpu.sync_copy(ids_hbm.at[pl.ds(seed_base, SPW)], ids0_v)

    def q0(i, c):
        ids0q_v[pl.ds(i * 16, 16)] = ids0_v[pl.ds(i * 16, 16)] // 4
        return c
    lax.fori_loop(0, SPW // 16, q0, 0)

    _gather_rows(adjq_hbm, ids0q_v, adj0_v, sem)

    def b1(i, c):
        mv = i * 16 + lax.iota(jnp.int32, 16)
        e = mv // S1
        nid = plsc.load_gather(ids0_v, [e])
        col = (nid % 4) * MAX_DEG + plsc.load_gather(p1_v, [mv % S1])
        val = plsc.load_gather(adj0_v, [e, col])
        ids1_v[pl.ds(i * 16, 16)] = val
        ids1q_v[pl.ds(i * 16, 16)] = val // 4
        return c
    lax.fori_loop(0, E1 // 16, b1, 0)

    for kc in range(E1 // CH):
        _gather_rows(adjq_hbm, ids1q_v.at[pl.ds(kc * CH, CH)], adjc_v, sem)

        def b2(i, c):
            mg = kc * CH * S2 + i * 16
            mv = mg + lax.iota(jnp.int32, 16)
            ent = mv // S2
            nid = plsc.load_gather(ids1_v, [ent])
            col = (nid % 4) * MAX_DEG + plsc.load_gather(p2_v, [mv % S2])
            ids2_v[pl.ds(mg, 16)] = plsc.load_gather(
                adjc_v, [ent - kc * CH, col])
            return c
        lax.fori_loop(0, CH * S2 // 16, b2, 0)

    pltpu.sync_copy(ids1_v, ids1_hbm.at[pl.ds(wid * E1, E1)])
    pltpu.sync_copy(ids2_v, ids2_hbm.at[pl.ds(wid * E2, E2)])


def _sc_ids(ids, adjq, p1, p2):
    mesh = plsc.VectorSubcoreMesh(core_axis_name="c", subcore_axis_name="s")
    i32 = jnp.int32
    k = pl.kernel(
        _ids_body,
        out_type=[jax.ShapeDtypeStruct((B * S1,), i32),
                  jax.ShapeDtypeStruct((B * S1 * S2,), i32)],
        mesh=mesh,
        compiler_params=pltpu.CompilerParams(needs_layout_passes=False),
        scratch_types=[
            pltpu.VMEM((SPW,), i32),
            pltpu.VMEM((SPW,), i32),
            pltpu.VMEM((32,), i32),
            pltpu.VMEM((16,), i32),
            pltpu.VMEM((SPW, 128), i32),
            pltpu.VMEM((E1,), i32),
            pltpu.VMEM((E1,), i32),
            pltpu.VMEM((CH, 128), i32),
            pltpu.VMEM((E2,), i32),
            pltpu.SemaphoreType.DMA,
        ],
    )
    return k(ids, adjq, p1, p2)


def _sc_body(ids_hbm, ids1_hbm, ids2_hbm, pxn_hbm,
             h0x_hbm, h0n_hbm, m1x_hbm, m1n_hbm,
             ids0_v, ids1_v, ids2_v,
             rows_v, rows2_v, h0x_v, h0n_v, m1x_v, m1n_v, sem, sem2):
    wid = _worker_id()
    seed_base = wid * SPW

    pltpu.sync_copy(ids_hbm.at[pl.ds(seed_base, SPW)], ids0_v)
    pltpu.sync_copy(ids1_hbm.at[pl.ds(wid * E1, E1)], ids1_v)
    pltpu.sync_copy(ids2_hbm.at[pl.ds(wid * E2, E2)], ids2_v)

    _gather_rows(pxn_hbm, ids0_v, rows_v.at[pl.ds(0, SPW)], sem)

    def pd(r, c):
        pxs, _ = _load_pair(rows_v, r)
        for l in range(LB):
            h0x_v[r, pl.ds(l * 16, 16)] = jnp.maximum(pxs[l], 0.0)
        return c
    lax.fori_loop(0, SPW, pd, 0)

    bufs = (rows_v, rows2_v)
    sems = (sem, sem2)

    def seed_mean2(buf, k):
        spc = CH // S1

        def sb(s, c):
            def eb(e, accs):
                row = s * S1 + e
                pxs, pns = _load_pair(buf, row)
                ax = tuple(a + jnp.maximum(v, 0.0)
                           for a, v in zip(accs[:LB], pxs))
                an = tuple(a + v for a, v in zip(accs[LB:], pns))
                return ax + an
            accs = lax.fori_loop(
                0, S1, eb, tuple(jnp.zeros((16,), jnp.float32)
                                 for _ in range(2 * LB)))
            srow = k * spc + s
            for l in range(LB):
                m1x_v[srow, pl.ds(l * 16, 16)] = accs[l] * (1.0 / S1)
                h0n_v[srow, pl.ds(l * 16, 16)] = jnp.maximum(
                    accs[LB + l] * (1.0 / S1), 0.0)
            return c
        lax.fori_loop(0, spc, sb, 0)

    nab = E1 // CH

    def issue_ab(k, b):
        _issue_gather(pxn_hbm, ids1_v.at[pl.ds(k * CH, CH)], bufs[b], sems[b])

    issue_ab(0, 0)
    for u in range(nab):
        b = u % 2
        if u + 1 < nab:
            issue_ab(u + 1, (u + 1) % 2)
        _drain_gather(pxn_hbm.at[pl.ds(0, CH)], bufs[b], sems[b])
        seed_mean2(bufs[b], u)

    def zinit(r, c):
        for l in range(LB):
            m1n_v[r, pl.ds(l * 16, 16)] = jnp.zeros((16,), jnp.float32)
        return c
    lax.fori_loop(0, SPW, zinit, 0)

    epc = CH // S2
    nch = E2 // CH

    def pc_start(k, b):
        koff = pl.multiple_of(k * CH, CH)
        _issue_gather(pxn_hbm, ids2_v.at[pl.ds(koff, CH)], bufs[b], sems[b])

    def pc_compute(buf, k):
        def eb(e, c2):
            rowb = e * S2
            eg = k * epc + e
            srow = eg // S1
            _, vs = _load_pair(buf, rowb)
            for r in range(1, S2):
                _, vr = _load_pair(buf, rowb + r)
                vs = [a + b2 for a, b2 in zip(vs, vr)]
            for l in range(LB):
                v = jnp.maximum(vs[l] * (1.0 / S2), 0.0) * (1.0 / S1)
                plsc.addupdate(m1n_v.at[srow, pl.ds(l * 16, 16)], v)
            return c2
        lax.fori_loop(0, epc, eb, 0)

    def drain(b):
        _drain_gather(pxn_hbm.at[pl.ds(0, CH)], bufs[b], sems[b])

    pc_start(0, 0)

    def pc_pair(kp, c):
        k0 = pl.multiple_of(kp * 2, 2)
        pc_start(k0 + 1, 1)
        drain(0)
        pc_compute(bufs[0], k0)

        @pl.when(kp < nch // 2 - 1)
        def _():
            pc_start(k0 + 2, 0)
        drain(1)
        pc_compute(bufs[1], k0 + 1)
        return c
    lax.fori_loop(0, nch // 2, pc_pair, 0)

    pltpu.sync_copy(h0x_v, h0x_hbm.at[pl.ds(seed_base, SPW)])
    pltpu.sync_copy(h0n_v, h0n_hbm.at[pl.ds(seed_base, SPW)])
    pltpu.sync_copy(m1x_v, m1x_hbm.at[pl.ds(seed_base, SPW)])
    pltpu.sync_copy(m1n_v, m1n_hbm.at[pl.ds(seed_base, SPW)])


def _sc_gather(ids, ids1, ids2, pxn):
    mesh = plsc.VectorSubcoreMesh(core_axis_name="c", subcore_axis_name="s")
    f32, i32 = jnp.float32, jnp.int32
    out = jax.ShapeDtypeStruct((B, H), f32)
    k = pl.kernel(
        _sc_body,
        out_type=[out, out, out, out],
        mesh=mesh,
        compiler_params=pltpu.CompilerParams(needs_layout_passes=False),
        scratch_types=[
            pltpu.VMEM((SPW,), i32),
            pltpu.VMEM((E1,), i32),
            pltpu.VMEM((E2,), i32),
            pltpu.VMEM((CH, H), jnp.uint32),
            pltpu.VMEM((CH, H), jnp.uint32),
            pltpu.VMEM((SPW, H), f32),
            pltpu.VMEM((SPW, H), f32),
            pltpu.VMEM((SPW, H), f32),
            pltpu.VMEM((SPW, H), f32),
            pltpu.SemaphoreType.DMA,
            pltpu.SemaphoreType.DMA,
        ],
    )
    return k(ids, ids1, ids2, pxn)


def _head_body(h0x, h0n, m1x, m1n, wx2, wn2, wfc, bx2, bn2, bfc, out):
    gx = (jnp.dot(h0x[...], wx2[:H, :], preferred_element_type=jnp.float32)
          + jnp.dot(h0n[...], wx2[H:, :], preferred_element_type=jnp.float32)
          + bx2[...])
    gn = (jnp.dot(m1x[...], wn2[:H, :], preferred_element_type=jnp.float32)
          + jnp.dot(m1n[...], wn2[H:, :], preferred_element_type=jnp.float32)
          + bn2[...])
    nrm = (jnp.sum(gx * gx, axis=1, keepdims=True)
           + jnp.sum(gn * gn, axis=1, keepdims=True))
    s = 1.0 / jnp.maximum(jnp.sqrt(nrm), 1e-12)
    out[...] = (jnp.dot(gx * s, wfc[:H, :], preferred_element_type=jnp.float32)
                + jnp.dot(gn * s, wfc[H:, :],
                          preferred_element_type=jnp.float32)
                + bfc[...])


def _head(h0x, h0n, m1x, m1n, wx2, wn2, wfc, bx2, bn2, bfc):
    return pl.pallas_call(
        _head_body,
        out_shape=jax.ShapeDtypeStruct((B, N_CLASSES), jnp.float32),
    )(h0x, h0n, m1x, m1n, wx2, wn2, wfc, bx2, bn2, bfc)


def kernel(ids, feats, adj, perm1, perm2, W_x1, b_x1, W_n1, b_n1,
           W_x2, b_x2, W_n2, b_n2, W_fc, b_fc):
    wcat = jnp.concatenate([W_x1, W_n1], axis=1)
    bcat = jnp.concatenate([b_x1, b_n1]).reshape(1, 2 * H)
    pxn = _project(feats, wcat, bcat)
    p1 = jnp.pad(perm1, (0, 32 - S1))
    p2 = jnp.pad(perm2, (0, 16 - S2))
    ids1, ids2 = _sc_ids(ids, adj.reshape(N_NODES // 4, 4 * MAX_DEG), p1, p2)
    h0x, h0n, m1x, m1n = _sc_gather(ids, ids1, ids2, pxn)
    return _head(h0x, h0n, m1x, m1n, W_x2, W_n2, W_fc,
                 b_x2.reshape(1, H), b_n2.reshape(1, H),
                 b_fc.reshape(1, N_CLASSES))


# speedup vs baseline: 4.2445x; 1.0024x over previous
import jax
import jax.numpy as jnp
from jax import lax
from jax.experimental import pallas as pl
from jax.experimental.pallas import tpu as pltpu
from jax.experimental.pallas import tpu_sc as plsc

N_NODES = 100000
MAX_DEG = 32
D_IN = 256
N_CLASSES = 40
B = 1024
S1, S2 = 25, 10
H = 128

NC, NS = 2, 16
NW = NC * NS
SPW = B // NW
E1 = SPW * S1
E2 = E1 * S2
CH = 200
LB = H // 16


def _bf16_bits(x):
    u = jax.lax.bitcast_convert_type(x, jnp.uint32)
    return (u + jnp.uint32(0x7FFF) + ((u >> 16) & jnp.uint32(1))) >> 16


def _proj_body(f_ref, w_ref, b_ref, pxn_ref):
    acc = jnp.dot(f_ref[...].astype(jnp.bfloat16), w_ref[...],
                  preferred_element_type=jnp.float32)
    acc = acc + b_ref[...]
    px = _bf16_bits(acc[:, :H])
    pn = _bf16_bits(acc[:, H:])
    pxn_ref[...] = px | (pn << 16)


def _project(feats, wcat, bcat):
    rows = 2000
    grid = N_NODES // rows
    return pl.pallas_call(
        _proj_body,
        grid=(grid,),
        in_specs=[
            pl.BlockSpec((rows, D_IN), lambda i: (i, 0)),
            pl.BlockSpec((D_IN, 2 * H), lambda i: (0, 0)),
            pl.BlockSpec((1, 2 * H), lambda i: (0, 0)),
        ],
        out_specs=pl.BlockSpec((rows, H), lambda i: (i, 0)),
        out_shape=jax.ShapeDtypeStruct((N_NODES, H), jnp.uint32),
    )(feats, wcat, bcat)


def _gather_rows(table_hbm, idx_ref, dst_ref, sem):
    pltpu.async_copy(table_hbm.at[idx_ref], dst_ref, sem).wait()


def _issue_gather(table_hbm, idx_ref, dst_ref, sem):
    pltpu.async_copy(table_hbm.at[idx_ref], dst_ref, sem)


def _drain_gather(dummy_hbm, dst_ref, sem):
    pltpu.make_async_copy(dummy_hbm, dst_ref, sem).wait()


def _worker_id():
    return lax.axis_index("s") * NC + lax.axis_index("c")


def _load_pair(buf, row):
    pxs, pns = [], []
    for blk in range(LB):
        w = buf[row, pl.ds(blk * 16, 16)]
        ab = plsc.bitcast(w, jnp.bfloat16)
        a, b = plsc.unpack(ab, format=plsc.PackFormat.INTERLEAVED)
        pxs.append(a)
        pns.append(b)
    return pxs, pns


def _ids_body(ids_hbm, adjq_hbm, p1_hbm, p2_hbm,
              ids1_hbm, ids2_hbm,
              ids0_v, ids0q_v, p1_v, p2_v, adj0_v, ids1_v, ids1q_v, adjc_v,
              ids2_v, sem):
    wid = _worker_id()
    seed_base = wid * SPW

    pltpu.sync_copy(p1_hbm, p1_v)
    pltpu.sync_copy(p2_hbm, p2_v)
    pltpu.sync_copy(ids_hbm.at[pl.ds(seed_base, SPW)], ids0_v)

    def q0(i, c):
        ids0q_v[pl.ds(i * 16, 16)] = ids0_v[pl.ds(i * 16, 16)] // 4
        return c
    lax.fori_loop(0, SPW // 16, q0, 0)

    _gather_rows(adjq_hbm, ids0q_v, adj0_v, sem)

    def b1(i, c):
        mv = i * 16 + lax.iota(jnp.int32, 16)
        e = mv // S1
        nid = plsc.load_gather(ids0_v, [e])
        col = (nid % 4) * MAX_DEG + plsc.load_gather(p1_v, [mv % S1])
        val = plsc.load_gather(adj0_v, [e, col])
        ids1_v[pl.ds(i * 16, 16)] = val
        ids1q_v[pl.ds(i * 16, 16)] = val // 4
        return c
    lax.fori_loop(0, E1 // 16, b1, 0)

    for kc in range(E1 // CH):
        _gather_rows(adjq_hbm, ids1q_v.at[pl.ds(kc * CH, CH)], adjc_v, sem)

        def b2(i, c):
            mg = kc * CH * S2 + i * 16
            mv = mg + lax.iota(jnp.int32, 16)
            ent = mv // S2
            nid = plsc.load_gather(ids1_v, [ent])
            col = (nid % 4) * MAX_DEG + plsc.load_gather(p2_v, [mv % S2])
            ids2_v[pl.ds(mg, 16)] = plsc.load_gather(
                adjc_v, [ent - kc * CH, col])
            return c
        lax.fori_loop(0, CH * S2 // 16, b2, 0)

    pltpu.sync_copy(ids1_v, ids1_hbm.at[pl.ds(wid * E1, E1)])
    pltpu.sync_copy(ids2_v, ids2_hbm.at[pl.ds(wid * E2, E2)])


def _sc_ids(ids, adjq, p1, p2):
    mesh = plsc.VectorSubcoreMesh(core_axis_name="c", subcore_axis_name="s")
    i32 = jnp.int32
    k = pl.kernel(
        _ids_body,
        out_type=[jax.ShapeDtypeStruct((B * S1,), i32),
                  jax.ShapeDtypeStruct((B * S1 * S2,), i32)],
        mesh=mesh,
        compiler_params=pltpu.CompilerParams(needs_layout_passes=False),
        scratch_types=[
            pltpu.VMEM((SPW,), i32),
            pltpu.VMEM((SPW,), i32),
            pltpu.VMEM((32,), i32),
            pltpu.VMEM((16,), i32),
            pltpu.VMEM((SPW, 128), i32),
            pltpu.VMEM((E1,), i32),
            pltpu.VMEM((E1,), i32),
            pltpu.VMEM((CH, 128), i32),
            pltpu.VMEM((E2,), i32),
            pltpu.SemaphoreType.DMA,
        ],
    )
    return k(ids, adjq, p1, p2)


def _sc_body(ids_hbm, ids1_hbm, ids2_hbm, pxn_hbm,
             h0x_hbm, h0n_hbm, m1x_hbm, m1n_hbm,
             ids0_v, ids1_v, ids2_v,
             rows_v, rows2_v, h0x_v, h0n_v, m1x_v, m1n_v, sem, sem2):
    wid = _worker_id()
    seed_base = wid * SPW

    pltpu.sync_copy(ids_hbm.at[pl.ds(seed_base, SPW)], ids0_v)
    pltpu.sync_copy(ids1_hbm.at[pl.ds(wid * E1, E1)], ids1_v)
    pltpu.sync_copy(ids2_hbm.at[pl.ds(wid * E2, E2)], ids2_v)

    _gather_rows(pxn_hbm, ids0_v, rows_v.at[pl.ds(0, SPW)], sem)

    def pd(r, c):
        pxs, _ = _load_pair(rows_v, r)
        for l in range(LB):
            h0x_v[r, pl.ds(l * 16, 16)] = jnp.maximum(pxs[l], 0.0)
        return c
    lax.fori_loop(0, SPW, pd, 0)

    bufs = (rows_v, rows2_v)
    sems = (sem, sem2)

    def seed_mean2(buf, k):
        spc = CH // S1

        def sb(s, c):
            def eb(e, accs):
                row = s * S1 + e
                pxs, pns = _load_pair(buf, row)
                ax = tuple(a + jnp.maximum(v, 0.0)
                           for a, v in zip(accs[:LB], pxs))
                an = tuple(a + v for a, v in zip(accs[LB:], pns))
                return ax + an
            accs = lax.fori_loop(
                0, S1, eb, tuple(jnp.zeros((16,), jnp.float32)
                                 for _ in range(2 * LB)))
            srow = k * spc + s
            for l in range(LB):
                m1x_v[srow, pl.ds(l * 16, 16)] = accs[l] * (1.0 / S1)
                h0n_v[srow, pl.ds(l * 16, 16)] = jnp.maximum(
                    accs[LB + l] * (1.0 / S1), 0.0)
            return c
        lax.fori_loop(0, spc, sb, 0)

    nab = E1 // CH

    def issue_ab(k, b):
        _issue_gather(pxn_hbm, ids1_v.at[pl.ds(k * CH, CH)], bufs[b], sems[b])

    issue_ab(0, 0)
    for u in range(nab):
        b = u % 2
        if u + 1 < nab:
            issue_ab(u + 1, (u + 1) % 2)
        _drain_gather(pxn_hbm.at[pl.ds(0, CH)], bufs[b], sems[b])
        seed_mean2(bufs[b], u)

    def zinit(r, c):
        for l in range(LB):
            m1n_v[r, pl.ds(l * 16, 16)] = jnp.zeros((16,), jnp.float32)
        return c
    lax.fori_loop(0, SPW, zinit, 0)

    epc = CH // S2
    nch = E2 // CH

    def pc_start(k, b):
        koff = pl.multiple_of(k * CH, CH)
        _issue_gather(pxn_hbm, ids2_v.at[pl.ds(koff, CH)], bufs[b], sems[b])

    def pc_compute(buf, k):
        def eb(e, c2):
            rowb = e * S2
            eg = k * epc + e
            srow = eg // S1
            _, vs = _load_pair(buf, rowb)
            for r in range(1, S2):
                _, vr = _load_pair(buf, rowb + r)
                vs = [a + b2 for a, b2 in zip(vs, vr)]
            for l in range(LB):
                v = jnp.maximum(vs[l] * (1.0 / S2), 0.0) * (1.0 / S1)
                plsc.addupdate(m1n_v.at[srow, pl.ds(l * 16, 16)], v)
            return c2
        lax.fori_loop(0, epc, eb, 0)

    def drain(b):
        _drain_gather(pxn_hbm.at[pl.ds(0, CH)], bufs[b], sems[b])

    pc_start(0, 0)

    def pc_pair(kp, c):
        k0 = pl.multiple_of(kp * 2, 2)
        pc_start(k0 + 1, 1)
        drain(0)
        pc_compute(bufs[0], k0)

        @pl.when(kp < nch // 2 - 1)
        def _():
            pc_start(k0 + 2, 0)
        drain(1)
        pc_compute(bufs[1], k0 + 1)
        return c
    lax.fori_loop(0, nch // 2, pc_pair, 0)

    pltpu.sync_copy(h0x_v, h0x_hbm.at[pl.ds(seed_base, SPW)])
    pltpu.sync_copy(h0n_v, h0n_hbm.at[pl.ds(seed_base, SPW)])
    pltpu.sync_copy(m1x_v, m1x_hbm.at[pl.ds(seed_base, SPW)])
    pltpu.sync_copy(m1n_v, m1n_hbm.at[pl.ds(seed_base, SPW)])


def _sc_gather(ids, ids1, ids2, pxn):
    mesh = plsc.VectorSubcoreMesh(core_axis_name="c", subcore_axis_name="s")
    f32, i32 = jnp.float32, jnp.int32
    out = jax.ShapeDtypeStruct((B, H), f32)
    k = pl.kernel(
        _sc_body,
        out_type=[out, out, out, out],
        mesh=mesh,
        compiler_params=pltpu.CompilerParams(needs_layout_passes=False),
        scratch_types=[
            pltpu.VMEM((SPW,), i32),
            pltpu.VMEM((E1,), i32),
            pltpu.VMEM((E2,), i32),
            pltpu.VMEM((CH, H), jnp.uint32),
            pltpu.VMEM((CH, H), jnp.uint32),
            pltpu.VMEM((SPW, H), f32),
            pltpu.VMEM((SPW, H), f32),
            pltpu.VMEM((SPW, H), f32),
            pltpu.VMEM((SPW, H), f32),
            pltpu.SemaphoreType.DMA,
            pltpu.SemaphoreType.DMA,
        ],
    )
    return k(ids, ids1, ids2, pxn)


def _head_body(h0x, h0n, m1x, m1n, wx2, wn2, wfc, bx2, bn2, bfc, out):
    gx = (jnp.dot(h0x[...], wx2[:H, :], preferred_element_type=jnp.float32)
          + jnp.dot(h0n[...], wx2[H:, :], preferred_element_type=jnp.float32)
          + bx2[...])
    gn = (jnp.dot(m1x[...], wn2[:H, :], preferred_element_type=jnp.float32)
          + jnp.dot(m1n[...], wn2[H:, :], preferred_element_type=jnp.float32)
          + bn2[...])
    nrm = (jnp.sum(gx * gx, axis=1, keepdims=True)
           + jnp.sum(gn * gn, axis=1, keepdims=True))
    s = 1.0 / jnp.maximum(jnp.sqrt(nrm), 1e-12)
    out[...] = (jnp.dot(gx * s, wfc[:H, :], preferred_element_type=jnp.float32)
                + jnp.dot(gn * s, wfc[H:, :],
                          preferred_element_type=jnp.float32)
                + bfc[...])


def _head(h0x, h0n, m1x, m1n, wx2, wn2, wfc, bx2, bn2, bfc):
    return pl.pallas_call(
        _head_body,
        out_shape=jax.ShapeDtypeStruct((B, N_CLASSES), jnp.float32),
    )(h0x, h0n, m1x, m1n, wx2, wn2, wfc, bx2, bn2, bfc)


def kernel(ids, feats, adj, perm1, perm2, W_x1, b_x1, W_n1, b_n1,
           W_x2, b_x2, W_n2, b_n2, W_fc, b_fc):
    wcat = jnp.concatenate([W_x1, W_n1], axis=1).astype(jnp.bfloat16)
    bcat = jnp.concatenate([b_x1, b_n1]).reshape(1, 2 * H)
    pxn = _project(feats, wcat, bcat)
    p1 = jnp.pad(perm1, (0, 32 - S1))
    p2 = jnp.pad(perm2, (0, 16 - S2))
    ids1, ids2 = _sc_ids(ids, adj.reshape(N_NODES // 4, 4 * MAX_DEG), p1, p2)
    h0x, h0n, m1x, m1n = _sc_gather(ids, ids1, ids2, pxn)
    return _head(h0x, h0n, m1x, m1n, W_x2, W_n2, W_fc,
                 b_x2.reshape(1, H), b_n2.reshape(1, H),
                 b_fc.reshape(1, N_CLASSES))


# speedup vs baseline: 4.3092x; 1.0152x over previous
import jax
import jax.numpy as jnp
from jax import lax
from jax.experimental import pallas as pl
from jax.experimental.pallas import tpu as pltpu
from jax.experimental.pallas import tpu_sc as plsc

N_NODES = 100000
MAX_DEG = 32
D_IN = 256
N_CLASSES = 40
B = 1024
S1, S2 = 25, 10
H = 128

NC, NS = 2, 16
NW = NC * NS
SPW = B // NW
E1 = SPW * S1
E2 = E1 * S2
CH = 400
CHA = 200
LB = H // 16


def _bf16_bits(x):
    u = jax.lax.bitcast_convert_type(x, jnp.uint32)
    return (u + jnp.uint32(0x7FFF) + ((u >> 16) & jnp.uint32(1))) >> 16


def _proj_body(f_ref, w_ref, b_ref, pxn_ref):
    acc = jnp.dot(f_ref[...].astype(jnp.bfloat16), w_ref[...],
                  preferred_element_type=jnp.float32)
    acc = acc + b_ref[...]
    px = _bf16_bits(acc[:, :H])
    pn = _bf16_bits(acc[:, H:])
    pxn_ref[...] = px | (pn << 16)


def _project(feats, wcat, bcat):
    rows = 2000
    grid = N_NODES // rows
    return pl.pallas_call(
        _proj_body,
        grid=(grid,),
        in_specs=[
            pl.BlockSpec((rows, D_IN), lambda i: (i, 0)),
            pl.BlockSpec((D_IN, 2 * H), lambda i: (0, 0)),
            pl.BlockSpec((1, 2 * H), lambda i: (0, 0)),
        ],
        out_specs=pl.BlockSpec((rows, H), lambda i: (i, 0)),
        out_shape=jax.ShapeDtypeStruct((N_NODES, H), jnp.uint32),
    )(feats, wcat, bcat)


def _gather_rows(table_hbm, idx_ref, dst_ref, sem):
    pltpu.async_copy(table_hbm.at[idx_ref], dst_ref, sem).wait()


def _issue_gather(table_hbm, idx_ref, dst_ref, sem):
    pltpu.async_copy(table_hbm.at[idx_ref], dst_ref, sem)


def _drain_gather(dummy_hbm, dst_ref, sem):
    pltpu.make_async_copy(dummy_hbm, dst_ref, sem).wait()


def _worker_id():
    return lax.axis_index("s") * NC + lax.axis_index("c")


def _load_pair(buf, row):
    pxs, pns = [], []
    for blk in range(LB):
        w = buf[row, pl.ds(blk * 16, 16)]
        pxs.append(plsc.bitcast(w << 16, jnp.float32))
        pns.append(plsc.bitcast(w & jnp.uint32(0xFFFF0000), jnp.float32))
    return pxs, pns


def _ids_body(ids_hbm, adjq_hbm, p1_hbm, p2_hbm,
              ids1_hbm, ids2_hbm,
              ids0_v, ids0q_v, p1_v, p2_v, adj0_v, ids1_v, ids1q_v, adjc_v,
              ids2_v, sem):
    wid = _worker_id()
    seed_base = wid * SPW

    pltpu.sync_copy(p1_hbm, p1_v)
    pltpu.sync_copy(p2_hbm, p2_v)
    pltpu.sync_copy(ids_hbm.at[pl.ds(seed_base, SPW)], ids0_v)

    def q0(i, c):
        ids0q_v[pl.ds(i * 16, 16)] = ids0_v[pl.ds(i * 16, 16)] // 4
        return c
    lax.fori_loop(0, SPW // 16, q0, 0)

    _gather_rows(adjq_hbm, ids0q_v, adj0_v, sem)

    def b1(i, c):
        mv = i * 16 + lax.iota(jnp.int32, 16)
        e = mv // S1
        nid = plsc.load_gather(ids0_v, [e])
        col = (nid % 4) * MAX_DEG + plsc.load_gather(p1_v, [mv % S1])
        val = plsc.load_gather(adj0_v, [e, col])
        ids1_v[pl.ds(i * 16, 16)] = val
        ids1q_v[pl.ds(i * 16, 16)] = val // 4
        return c
    lax.fori_loop(0, E1 // 16, b1, 0)

    for kc in range(E1 // CHA):
        _gather_rows(adjq_hbm, ids1q_v.at[pl.ds(kc * CHA, CHA)], adjc_v, sem)

        def b2(i, c):
            mg = kc * CHA * S2 + i * 16
            mv = mg + lax.iota(jnp.int32, 16)
            ent = mv // S2
            nid = plsc.load_gather(ids1_v, [ent])
            col = (nid % 4) * MAX_DEG + plsc.load_gather(p2_v, [mv % S2])
            ids2_v[pl.ds(mg, 16)] = plsc.load_gather(
                adjc_v, [ent - kc * CHA, col])
            return c
        lax.fori_loop(0, CHA * S2 // 16, b2, 0)

    pltpu.sync_copy(ids1_v, ids1_hbm.at[pl.ds(wid * E1, E1)])
    pltpu.sync_copy(ids2_v, ids2_hbm.at[pl.ds(wid * E2, E2)])


def _sc_ids(ids, adjq, p1, p2):
    mesh = plsc.VectorSubcoreMesh(core_axis_name="c", subcore_axis_name="s")
    i32 = jnp.int32
    k = pl.kernel(
        _ids_body,
        out_type=[jax.ShapeDtypeStruct((B * S1,), i32),
                  jax.ShapeDtypeStruct((B * S1 * S2,), i32)],
        mesh=mesh,
        compiler_params=pltpu.CompilerParams(needs_layout_passes=False),
        scratch_types=[
            pltpu.VMEM((SPW,), i32),
            pltpu.VMEM((SPW,), i32),
            pltpu.VMEM((32,), i32),
            pltpu.VMEM((16,), i32),
            pltpu.VMEM((SPW, 128), i32),
            pltpu.VMEM((E1,), i32),
            pltpu.VMEM((E1,), i32),
            pltpu.VMEM((CHA, 128), i32),
            pltpu.VMEM((E2,), i32),
            pltpu.SemaphoreType.DMA,
        ],
    )
    return k(ids, adjq, p1, p2)


def _sc_body(ids_hbm, ids1_hbm, ids2_hbm, pxn_hbm,
             h0x_hbm, h0n_hbm, m1x_hbm, m1n_hbm,
             ids0_v, ids1_v, ids2_v,
             rows_v, rows2_v, h0x_v, h0n_v, m1x_v, m1n_v, sem, sem2):
    wid = _worker_id()
    seed_base = wid * SPW

    pltpu.sync_copy(ids_hbm.at[pl.ds(seed_base, SPW)], ids0_v)
    pltpu.sync_copy(ids1_hbm.at[pl.ds(wid * E1, E1)], ids1_v)
    pltpu.sync_copy(ids2_hbm.at[pl.ds(wid * E2, E2)], ids2_v)

    _gather_rows(pxn_hbm, ids0_v, rows_v.at[pl.ds(0, SPW)], sem)

    def pd(r, c):
        pxs, _ = _load_pair(rows_v, r)
        for l in range(LB):
            h0x_v[r, pl.ds(l * 16, 16)] = jnp.maximum(pxs[l], 0.0)
        return c
    lax.fori_loop(0, SPW, pd, 0)

    bufs = (rows_v, rows2_v)
    sems = (sem, sem2)

    def seed_mean2(buf, k):
        spc = CH // S1

        def sb(s, c):
            def eb(e, accs):
                row = s * S1 + e
                pxs, pns = _load_pair(buf, row)
                ax = tuple(a + jnp.maximum(v, 0.0)
                           for a, v in zip(accs[:LB], pxs))
                an = tuple(a + v for a, v in zip(accs[LB:], pns))
                return ax + an
            accs = lax.fori_loop(
                0, S1, eb, tuple(jnp.zeros((16,), jnp.float32)
                                 for _ in range(2 * LB)))
            srow = k * spc + s
            for l in range(LB):
                m1x_v[srow, pl.ds(l * 16, 16)] = accs[l] * (1.0 / S1)
                h0n_v[srow, pl.ds(l * 16, 16)] = jnp.maximum(
                    accs[LB + l] * (1.0 / S1), 0.0)
            return c
        lax.fori_loop(0, spc, sb, 0)

    nab = E1 // CH

    def issue_ab(k, b):
        _issue_gather(pxn_hbm, ids1_v.at[pl.ds(k * CH, CH)], bufs[b], sems[b])

    issue_ab(0, 0)
    for u in range(nab):
        b = u % 2
        if u + 1 < nab:
            issue_ab(u + 1, (u + 1) % 2)
        _drain_gather(pxn_hbm.at[pl.ds(0, CH)], bufs[b], sems[b])
        seed_mean2(bufs[b], u)

    def zinit(r, c):
        for l in range(LB):
            m1n_v[r, pl.ds(l * 16, 16)] = jnp.zeros((16,), jnp.float32)
        return c
    lax.fori_loop(0, SPW, zinit, 0)

    epc = CH // S2
    nch = E2 // CH

    def pc_start(k, b):
        koff = pl.multiple_of(k * CH, CH)
        _issue_gather(pxn_hbm, ids2_v.at[pl.ds(koff, CH)], bufs[b], sems[b])

    def pc_compute(buf, k):
        def eb(e, c2):
            rowb = e * S2
            eg = k * epc + e
            srow = eg // S1
            _, vs = _load_pair(buf, rowb)
            for r in range(1, S2):
                _, vr = _load_pair(buf, rowb + r)
                vs = [a + b2 for a, b2 in zip(vs, vr)]
            for l in range(LB):
                v = jnp.maximum(vs[l] * (1.0 / S2), 0.0) * (1.0 / S1)
                plsc.addupdate(m1n_v.at[srow, pl.ds(l * 16, 16)], v)
            return c2
        lax.fori_loop(0, epc, eb, 0)

    def drain(b):
        _drain_gather(pxn_hbm.at[pl.ds(0, CH)], bufs[b], sems[b])

    pc_start(0, 0)

    def pc_pair(kp, c):
        k0 = pl.multiple_of(kp * 2, 2)
        pc_start(k0 + 1, 1)
        drain(0)
        pc_compute(bufs[0], k0)

        @pl.when(kp < nch // 2 - 1)
        def _():
            pc_start(k0 + 2, 0)
        drain(1)
        pc_compute(bufs[1], k0 + 1)
        return c
    lax.fori_loop(0, nch // 2, pc_pair, 0)

    pltpu.sync_copy(h0x_v, h0x_hbm.at[pl.ds(seed_base, SPW)])
    pltpu.sync_copy(h0n_v, h0n_hbm.at[pl.ds(seed_base, SPW)])
    pltpu.sync_copy(m1x_v, m1x_hbm.at[pl.ds(seed_base, SPW)])
    pltpu.sync_copy(m1n_v, m1n_hbm.at[pl.ds(seed_base, SPW)])


def _sc_gather(ids, ids1, ids2, pxn):
    mesh = plsc.VectorSubcoreMesh(core_axis_name="c", subcore_axis_name="s")
    f32, i32 = jnp.float32, jnp.int32
    out = jax.ShapeDtypeStruct((B, H), f32)
    k = pl.kernel(
        _sc_body,
        out_type=[out, out, out, out],
        mesh=mesh,
        compiler_params=pltpu.CompilerParams(needs_layout_passes=False),
        scratch_types=[
            pltpu.VMEM((SPW,), i32),
            pltpu.VMEM((E1,), i32),
            pltpu.VMEM((E2,), i32),
            pltpu.VMEM((CH, H), jnp.uint32),
            pltpu.VMEM((CH, H), jnp.uint32),
            pltpu.VMEM((SPW, H), f32),
            pltpu.VMEM((SPW, H), f32),
            pltpu.VMEM((SPW, H), f32),
            pltpu.VMEM((SPW, H), f32),
            pltpu.SemaphoreType.DMA,
            pltpu.SemaphoreType.DMA,
        ],
    )
    return k(ids, ids1, ids2, pxn)


def _head_body(h0x, h0n, m1x, m1n, wx2, wn2, wfc, bx2, bn2, bfc, out):
    gx = (jnp.dot(h0x[...], wx2[:H, :], preferred_element_type=jnp.float32)
          + jnp.dot(h0n[...], wx2[H:, :], preferred_element_type=jnp.float32)
          + bx2[...])
    gn = (jnp.dot(m1x[...], wn2[:H, :], preferred_element_type=jnp.float32)
          + jnp.dot(m1n[...], wn2[H:, :], preferred_element_type=jnp.float32)
          + bn2[...])
    nrm = (jnp.sum(gx * gx, axis=1, keepdims=True)
           + jnp.sum(gn * gn, axis=1, keepdims=True))
    s = 1.0 / jnp.maximum(jnp.sqrt(nrm), 1e-12)
    out[...] = (jnp.dot(gx * s, wfc[:H, :], preferred_element_type=jnp.float32)
                + jnp.dot(gn * s, wfc[H:, :],
                          preferred_element_type=jnp.float32)
                + bfc[...])


def _head(h0x, h0n, m1x, m1n, wx2, wn2, wfc, bx2, bn2, bfc):
    return pl.pallas_call(
        _head_body,
        out_shape=jax.ShapeDtypeStruct((B, N_CLASSES), jnp.float32),
    )(h0x, h0n, m1x, m1n, wx2, wn2, wfc, bx2, bn2, bfc)


def kernel(ids, feats, adj, perm1, perm2, W_x1, b_x1, W_n1, b_n1,
           W_x2, b_x2, W_n2, b_n2, W_fc, b_fc):
    wcat = jnp.concatenate([W_x1, W_n1], axis=1).astype(jnp.bfloat16)
    bcat = jnp.concatenate([b_x1, b_n1]).reshape(1, 2 * H)
    pxn = _project(feats, wcat, bcat)
    p1 = jnp.pad(perm1, (0, 32 - S1))
    p2 = jnp.pad(perm2, (0, 16 - S2))
    ids1, ids2 = _sc_ids(ids, adj.reshape(N_NODES // 4, 4 * MAX_DEG), p1, p2)
    h0x, h0n, m1x, m1n = _sc_gather(ids, ids1, ids2, pxn)
    return _head(h0x, h0n, m1x, m1n, W_x2, W_n2, W_fc,
                 b_x2.reshape(1, H), b_n2.reshape(1, H),
                 b_fc.reshape(1, N_CLASSES))


# speedup vs baseline: 4.7559x; 1.1037x over previous
import jax
import jax.numpy as jnp
from jax import lax
from jax.experimental import pallas as pl
from jax.experimental.pallas import tpu as pltpu
from jax.experimental.pallas import tpu_sc as plsc

N_NODES = 100000
MAX_DEG = 32
D_IN = 256
N_CLASSES = 40
B = 1024
S1, S2 = 25, 10
H = 128

NC, NS = 2, 16
NW = NC * NS
SPW = B // NW
E1 = SPW * S1
E2 = E1 * S2
CH = 400
CHA = 200
LB = H // 16


def _bf16_bits(x):
    u = jax.lax.bitcast_convert_type(x, jnp.uint32)
    return (u + jnp.uint32(0x7FFF) + ((u >> 16) & jnp.uint32(1))) >> 16


def _proj_body(f_ref, w_ref, b_ref, pxn_ref):
    acc = jnp.dot(f_ref[...].astype(jnp.bfloat16), w_ref[...],
                  preferred_element_type=jnp.float32)
    acc = acc + b_ref[...]
    px = _bf16_bits(acc[:, :H])
    pn = _bf16_bits(acc[:, H:])
    pxn_ref[...] = px | (pn << 16)


def _project(feats, wcat, bcat):
    rows = 4000
    grid = N_NODES // rows
    return pl.pallas_call(
        _proj_body,
        grid=(grid,),
        in_specs=[
            pl.BlockSpec((rows, D_IN), lambda i: (i, 0)),
            pl.BlockSpec((D_IN, 2 * H), lambda i: (0, 0)),
            pl.BlockSpec((1, 2 * H), lambda i: (0, 0)),
        ],
        out_specs=pl.BlockSpec((rows, H), lambda i: (i, 0)),
        out_shape=jax.ShapeDtypeStruct((N_NODES, H), jnp.uint32),
    )(feats, wcat, bcat)


def _gather_rows(table_hbm, idx_ref, dst_ref, sem):
    pltpu.async_copy(table_hbm.at[idx_ref], dst_ref, sem).wait()


def _issue_gather(table_hbm, idx_ref, dst_ref, sem):
    pltpu.async_copy(table_hbm.at[idx_ref], dst_ref, sem)


def _drain_gather(dummy_hbm, dst_ref, sem):
    pltpu.make_async_copy(dummy_hbm, dst_ref, sem).wait()


def _worker_id():
    return lax.axis_index("s") * NC + lax.axis_index("c")


def _load_pair(buf, row):
    pxs, pns = [], []
    for blk in range(LB):
        w = buf[row, pl.ds(blk * 16, 16)]
        pxs.append(plsc.bitcast(w << 16, jnp.float32))
        pns.append(plsc.bitcast(w & jnp.uint32(0xFFFF0000), jnp.float32))
    return pxs, pns


def _ids_body(ids_hbm, adjq_hbm, p1_hbm, p2_hbm,
              ids1_hbm, ids2_hbm,
              ids0_v, ids0q_v, p1_v, p2_v, adj0_v, ids1_v, ids1q_v, adjc_v,
              ids2_v, sem):
    wid = _worker_id()
    seed_base = wid * SPW

    pltpu.sync_copy(p1_hbm, p1_v)
    pltpu.sync_copy(p2_hbm, p2_v)
    pltpu.sync_copy(ids_hbm.at[pl.ds(seed_base, SPW)], ids0_v)

    def q0(i, c):
        ids0q_v[pl.ds(i * 16, 16)] = ids0_v[pl.ds(i * 16, 16)] // 4
        return c
    lax.fori_loop(0, SPW // 16, q0, 0)

    _gather_rows(adjq_hbm, ids0q_v, adj0_v, sem)

    def b1(i, c):
        mv = i * 16 + lax.iota(jnp.int32, 16)
        e = mv // S1
        nid = plsc.load_gather(ids0_v, [e])
        col = (nid % 4) * MAX_DEG + plsc.load_gather(p1_v, [mv % S1])
        val = plsc.load_gather(adj0_v, [e, col])
        ids1_v[pl.ds(i * 16, 16)] = val
        ids1q_v[pl.ds(i * 16, 16)] = val // 4
        return c
    lax.fori_loop(0, E1 // 16, b1, 0)

    for kc in range(E1 // CHA):
        _gather_rows(adjq_hbm, ids1q_v.at[pl.ds(kc * CHA, CHA)], adjc_v, sem)

        def b2(i, c):
            mg = kc * CHA * S2 + i * 16
            mv = mg + lax.iota(jnp.int32, 16)
            ent = mv // S2
            nid = plsc.load_gather(ids1_v, [ent])
            col = (nid % 4) * MAX_DEG + plsc.load_gather(p2_v, [mv % S2])
            ids2_v[pl.ds(mg, 16)] = plsc.load_gather(
                adjc_v, [ent - kc * CHA, col])
            return c
        lax.fori_loop(0, CHA * S2 // 16, b2, 0)

    pltpu.sync_copy(ids1_v, ids1_hbm.at[pl.ds(wid * E1, E1)])
    pltpu.sync_copy(ids2_v, ids2_hbm.at[pl.ds(wid * E2, E2)])


def _sc_ids(ids, adjq, p1, p2):
    mesh = plsc.VectorSubcoreMesh(core_axis_name="c", subcore_axis_name="s")
    i32 = jnp.int32
    k = pl.kernel(
        _ids_body,
        out_type=[jax.ShapeDtypeStruct((B * S1,), i32),
                  jax.ShapeDtypeStruct((B * S1 * S2,), i32)],
        mesh=mesh,
        compiler_params=pltpu.CompilerParams(needs_layout_passes=False),
        scratch_types=[
            pltpu.VMEM((SPW,), i32),
            pltpu.VMEM((SPW,), i32),
            pltpu.VMEM((32,), i32),
            pltpu.VMEM((16,), i32),
            pltpu.VMEM((SPW, 128), i32),
            pltpu.VMEM((E1,), i32),
            pltpu.VMEM((E1,), i32),
            pltpu.VMEM((CHA, 128), i32),
            pltpu.VMEM((E2,), i32),
            pltpu.SemaphoreType.DMA,
        ],
    )
    return k(ids, adjq, p1, p2)


def _sc_body(ids_hbm, ids1_hbm, ids2_hbm, pxn_hbm,
             h0x_hbm, h0n_hbm, m1x_hbm, m1n_hbm,
             ids0_v, ids1_v, ids2_v,
             rows_v, rows2_v, h0x_v, h0n_v, m1x_v, m1n_v, sem, sem2):
    wid = _worker_id()
    seed_base = wid * SPW

    pltpu.sync_copy(ids_hbm.at[pl.ds(seed_base, SPW)], ids0_v)
    pltpu.sync_copy(ids1_hbm.at[pl.ds(wid * E1, E1)], ids1_v)
    pltpu.sync_copy(ids2_hbm.at[pl.ds(wid * E2, E2)], ids2_v)

    bufs = (rows_v, rows2_v)
    sems = (sem, sem2)

    def zinit(r, c):
        for l in range(LB):
            m1n_v[r, pl.ds(l * 16, 16)] = jnp.zeros((16,), jnp.float32)
        return c
    lax.fori_loop(0, SPW, zinit, 0)

    def seed_mean2(buf, k):
        spc = CH // S1

        def sb(s, c):
            def eb(e, accs):
                row = s * S1 + e
                pxs, pns = _load_pair(buf, row)
                ax = tuple(a + jnp.maximum(v, 0.0)
                           for a, v in zip(accs[:LB], pxs))
                an = tuple(a + v for a, v in zip(accs[LB:], pns))
                return ax + an
            accs = lax.fori_loop(
                0, S1, eb, tuple(jnp.zeros((16,), jnp.float32)
                                 for _ in range(2 * LB)))
            srow = k * spc + s
            for l in range(LB):
                m1x_v[srow, pl.ds(l * 16, 16)] = accs[l] * (1.0 / S1)
                h0n_v[srow, pl.ds(l * 16, 16)] = jnp.maximum(
                    accs[LB + l] * (1.0 / S1), 0.0)
            return c
        lax.fori_loop(0, spc, sb, 0)

    epc = CH // S2
    nch = E2 // CH

    def pc_start(k, b):
        koff = pl.multiple_of(k * CH, CH)
        _issue_gather(pxn_hbm, ids2_v.at[pl.ds(koff, CH)], bufs[b], sems[b])

    def pc_compute(buf, k):
        def eb(e, c2):
            rowb = e * S2
            eg = k * epc + e
            srow = eg // S1
            _, vs = _load_pair(buf, rowb)
            for r in range(1, S2):
                _, vr = _load_pair(buf, rowb + r)
                vs = [a + b2 for a, b2 in zip(vs, vr)]
            for l in range(LB):
                v = jnp.maximum(vs[l] * (1.0 / S2), 0.0) * (1.0 / S1)
                plsc.addupdate(m1n_v.at[srow, pl.ds(l * 16, 16)], v)
            return c2
        lax.fori_loop(0, epc, eb, 0)

    def drain(b):
        _drain_gather(pxn_hbm.at[pl.ds(0, CH)], bufs[b], sems[b])

    dsub = rows_v.at[pl.ds(0, SPW)]

    _issue_gather(pxn_hbm, ids0_v, dsub, sems[0])
    _issue_gather(pxn_hbm, ids1_v.at[pl.ds(0, CH)], bufs[1], sems[1])
    _drain_gather(pxn_hbm.at[pl.ds(0, SPW)], dsub, sems[0])

    def pd(r, c):
        pxs, _ = _load_pair(rows_v, r)
        for l in range(LB):
            h0x_v[r, pl.ds(l * 16, 16)] = jnp.maximum(pxs[l], 0.0)
        return c
    lax.fori_loop(0, SPW, pd, 0)

    _issue_gather(pxn_hbm, ids1_v.at[pl.ds(CH, CH)], bufs[0], sems[0])
    drain(1)
    seed_mean2(bufs[1], 0)
    pc_start(0, 1)
    drain(0)
    seed_mean2(bufs[0], 1)
    pc_start(1, 0)

    def pc_pair(kp, c):
        k0 = pl.multiple_of(kp * 2, 2)
        drain(1)
        pc_compute(bufs[1], k0)

        @pl.when(kp < nch // 2 - 1)
        def _():
            pc_start(k0 + 2, 1)
        drain(0)
        pc_compute(bufs[0], k0 + 1)

        @pl.when(kp < nch // 2 - 1)
        def _():
            pc_start(k0 + 3, 0)
        return c
    lax.fori_loop(0, nch // 2, pc_pair, 0)

    pltpu.sync_copy(h0x_v, h0x_hbm.at[pl.ds(seed_base, SPW)])
    pltpu.sync_copy(h0n_v, h0n_hbm.at[pl.ds(seed_base, SPW)])
    pltpu.sync_copy(m1x_v, m1x_hbm.at[pl.ds(seed_base, SPW)])
    pltpu.sync_copy(m1n_v, m1n_hbm.at[pl.ds(seed_base, SPW)])


def _sc_gather(ids, ids1, ids2, pxn):
    mesh = plsc.VectorSubcoreMesh(core_axis_name="c", subcore_axis_name="s")
    f32, i32 = jnp.float32, jnp.int32
    out = jax.ShapeDtypeStruct((B, H), f32)
    k = pl.kernel(
        _sc_body,
        out_type=[out, out, out, out],
        mesh=mesh,
        compiler_params=pltpu.CompilerParams(needs_layout_passes=False),
        scratch_types=[
            pltpu.VMEM((SPW,), i32),
            pltpu.VMEM((E1,), i32),
            pltpu.VMEM((E2,), i32),
            pltpu.VMEM((CH, H), jnp.uint32),
            pltpu.VMEM((CH, H), jnp.uint32),
            pltpu.VMEM((SPW, H), f32),
            pltpu.VMEM((SPW, H), f32),
            pltpu.VMEM((SPW, H), f32),
            pltpu.VMEM((SPW, H), f32),
            pltpu.SemaphoreType.DMA,
            pltpu.SemaphoreType.DMA,
        ],
    )
    return k(ids, ids1, ids2, pxn)


def _head_body(h0x, h0n, m1x, m1n, wx2, wn2, wfc, bx2, bn2, bfc, out):
    gx = (jnp.dot(h0x[...], wx2[:H, :], preferred_element_type=jnp.float32)
          + jnp.dot(h0n[...], wx2[H:, :], preferred_element_type=jnp.float32)
          + bx2[...])
    gn = (jnp.dot(m1x[...], wn2[:H, :], preferred_element_type=jnp.float32)
          + jnp.dot(m1n[...], wn2[H:, :], preferred_element_type=jnp.float32)
          + bn2[...])
    nrm = (jnp.sum(gx * gx, axis=1, keepdims=True)
           + jnp.sum(gn * gn, axis=1, keepdims=True))
    s = 1.0 / jnp.maximum(jnp.sqrt(nrm), 1e-12)
    out[...] = (jnp.dot(gx * s, wfc[:H, :], preferred_element_type=jnp.float32)
                + jnp.dot(gn * s, wfc[H:, :],
                          preferred_element_type=jnp.float32)
                + bfc[...])


def _head(h0x, h0n, m1x, m1n, wx2, wn2, wfc, bx2, bn2, bfc):
    return pl.pallas_call(
        _head_body,
        out_shape=jax.ShapeDtypeStruct((B, N_CLASSES), jnp.float32),
    )(h0x, h0n, m1x, m1n, wx2, wn2, wfc, bx2, bn2, bfc)


def kernel(ids, feats, adj, perm1, perm2, W_x1, b_x1, W_n1, b_n1,
           W_x2, b_x2, W_n2, b_n2, W_fc, b_fc):
    wcat = jnp.concatenate([W_x1, W_n1], axis=1).astype(jnp.bfloat16)
    bcat = jnp.concatenate([b_x1, b_n1]).reshape(1, 2 * H)
    pxn = _project(feats, wcat, bcat)
    p1 = jnp.pad(perm1, (0, 32 - S1))
    p2 = jnp.pad(perm2, (0, 16 - S2))
    ids1, ids2 = _sc_ids(ids, adj.reshape(N_NODES // 4, 4 * MAX_DEG), p1, p2)
    h0x, h0n, m1x, m1n = _sc_gather(ids, ids1, ids2, pxn)
    return _head(h0x, h0n, m1x, m1n, W_x2, W_n2, W_fc,
                 b_x2.reshape(1, H), b_n2.reshape(1, H),
                 b_fc.reshape(1, N_CLASSES))


# speedup vs baseline: 4.8754x; 1.0251x over previous
import jax
import jax.numpy as jnp
from jax import lax
from jax.experimental import pallas as pl
from jax.experimental.pallas import tpu as pltpu
from jax.experimental.pallas import tpu_sc as plsc

N_NODES = 100000
MAX_DEG = 32
D_IN = 256
N_CLASSES = 40
B = 1024
S1, S2 = 25, 10
H = 128

NC, NS = 2, 16
NW = NC * NS
SPW = B // NW
E1 = SPW * S1
E2 = E1 * S2
CH = 400
CHA = 200
LB = H // 16


def _bf16_bits(x):
    u = jax.lax.bitcast_convert_type(x, jnp.uint32)
    return (u + jnp.uint32(0x7FFF) + ((u >> 16) & jnp.uint32(1))) >> 16


def _proj_body(f_ref, w_ref, b_ref, pxn_ref):
    acc = jnp.dot(f_ref[...].astype(jnp.bfloat16), w_ref[...],
                  preferred_element_type=jnp.float32)
    acc = acc + b_ref[...]
    px = _bf16_bits(acc[:, :H])
    pn = _bf16_bits(acc[:, H:])
    pxn_ref[...] = px | (pn << 16)


def _project(feats, wcat, bcat):
    rows = 10000
    grid = N_NODES // rows
    return pl.pallas_call(
        _proj_body,
        grid=(grid,),
        in_specs=[
            pl.BlockSpec((rows, D_IN), lambda i: (i, 0)),
            pl.BlockSpec((D_IN, 2 * H), lambda i: (0, 0)),
            pl.BlockSpec((1, 2 * H), lambda i: (0, 0)),
        ],
        out_specs=pl.BlockSpec((rows, H), lambda i: (i, 0)),
        out_shape=jax.ShapeDtypeStruct((N_NODES, H), jnp.uint32),
    )(feats, wcat, bcat)


def _gather_rows(table_hbm, idx_ref, dst_ref, sem):
    pltpu.async_copy(table_hbm.at[idx_ref], dst_ref, sem).wait()


def _issue_gather(table_hbm, idx_ref, dst_ref, sem):
    pltpu.async_copy(table_hbm.at[idx_ref], dst_ref, sem)


def _drain_gather(dummy_hbm, dst_ref, sem):
    pltpu.make_async_copy(dummy_hbm, dst_ref, sem).wait()


def _worker_id():
    return lax.axis_index("s") * NC + lax.axis_index("c")


def _load_pair(buf, row):
    pxs, pns = [], []
    for blk in range(LB):
        w = buf[row, pl.ds(blk * 16, 16)]
        pxs.append(plsc.bitcast(w << 16, jnp.float32))
        pns.append(plsc.bitcast(w & jnp.uint32(0xFFFF0000), jnp.float32))
    return pxs, pns


def _ids_body(ids_hbm, adjq_hbm, p1_hbm, p2_hbm,
              ids1_hbm, ids2_hbm,
              ids0_v, ids0q_v, p1_v, p2_v, adj0_v, ids1_v, ids1q_v, adjc_v,
              ids2_v, sem):
    wid = _worker_id()
    seed_base = wid * SPW

    pltpu.sync_copy(p1_hbm, p1_v)
    pltpu.sync_copy(p2_hbm, p2_v)
    pltpu.sync_copy(ids_hbm.at[pl.ds(seed_base, SPW)], ids0_v)

    def q0(i, c):
        ids0q_v[pl.ds(i * 16, 16)] = ids0_v[pl.ds(i * 16, 16)] // 4
        return c
    lax.fori_loop(0, SPW // 16, q0, 0)

    _gather_rows(adjq_hbm, ids0q_v, adj0_v, sem)

    def b1(i, c):
        mv = i * 16 + lax.iota(jnp.int32, 16)
        e = mv // S1
        nid = plsc.load_gather(ids0_v, [e])
        col = (nid % 4) * MAX_DEG + plsc.load_gather(p1_v, [mv % S1])
        val = plsc.load_gather(adj0_v, [e, col])
        ids1_v[pl.ds(i * 16, 16)] = val
        ids1q_v[pl.ds(i * 16, 16)] = val // 4
        return c
    lax.fori_loop(0, E1 // 16, b1, 0)

    for kc in range(E1 // CHA):
        _gather_rows(adjq_hbm, ids1q_v.at[pl.ds(kc * CHA, CHA)], adjc_v, sem)

        def b2(i, c):
            mg = kc * CHA * S2 + i * 16
            mv = mg + lax.iota(jnp.int32, 16)
            ent = mv // S2
            nid = plsc.load_gather(ids1_v, [ent])
            col = (nid % 4) * MAX_DEG + plsc.load_gather(p2_v, [mv % S2])
            ids2_v[pl.ds(mg, 16)] = plsc.load_gather(
                adjc_v, [ent - kc * CHA, col])
            return c
        lax.fori_loop(0, CHA * S2 // 16, b2, 0)

    pltpu.sync_copy(ids1_v, ids1_hbm.at[pl.ds(wid * E1, E1)])
    pltpu.sync_copy(ids2_v, ids2_hbm.at[pl.ds(wid * E2, E2)])


def _sc_ids(ids, adjq, p1, p2):
    mesh = plsc.VectorSubcoreMesh(core_axis_name="c", subcore_axis_name="s")
    i32 = jnp.int32
    k = pl.kernel(
        _ids_body,
        out_type=[jax.ShapeDtypeStruct((B * S1,), i32),
                  jax.ShapeDtypeStruct((B * S1 * S2,), i32)],
        mesh=mesh,
        compiler_params=pltpu.CompilerParams(needs_layout_passes=False),
        scratch_types=[
            pltpu.VMEM((SPW,), i32),
            pltpu.VMEM((SPW,), i32),
            pltpu.VMEM((32,), i32),
            pltpu.VMEM((16,), i32),
            pltpu.VMEM((SPW, 128), i32),
            pltpu.VMEM((E1,), i32),
            pltpu.VMEM((E1,), i32),
            pltpu.VMEM((CHA, 128), i32),
            pltpu.VMEM((E2,), i32),
            pltpu.SemaphoreType.DMA,
        ],
    )
    return k(ids, adjq, p1, p2)


def _sc_body(ids_hbm, ids1_hbm, ids2_hbm, pxn_hbm,
             h0x_hbm, h0n_hbm, m1x_hbm, m1n_hbm,
             ids0_v, ids1_v, ids2_v,
             rows_v, rows2_v, h0x_v, h0n_v, m1x_v, m1n_v, sem, sem2):
    wid = _worker_id()
    seed_base = wid * SPW

    pltpu.sync_copy(ids_hbm.at[pl.ds(seed_base, SPW)], ids0_v)
    pltpu.sync_copy(ids1_hbm.at[pl.ds(wid * E1, E1)], ids1_v)
    pltpu.sync_copy(ids2_hbm.at[pl.ds(wid * E2, E2)], ids2_v)

    bufs = (rows_v, rows2_v)
    sems = (sem, sem2)

    def zinit(r, c):
        for l in range(LB):
            m1n_v[r, pl.ds(l * 16, 16)] = jnp.zeros((16,), jnp.float32)
        return c
    lax.fori_loop(0, SPW, zinit, 0)

    def seed_mean2(buf, k):
        spc = CH // S1

        def sb(s, c):
            def eb(e, accs):
                row = s * S1 + e
                pxs, pns = _load_pair(buf, row)
                ax = tuple(a + jnp.maximum(v, 0.0)
                           for a, v in zip(accs[:LB], pxs))
                an = tuple(a + v for a, v in zip(accs[LB:], pns))
                return ax + an
            accs = lax.fori_loop(
                0, S1, eb, tuple(jnp.zeros((16,), jnp.float32)
                                 for _ in range(2 * LB)))
            srow = k * spc + s
            for l in range(LB):
                m1x_v[srow, pl.ds(l * 16, 16)] = accs[l] * (1.0 / S1)
                h0n_v[srow, pl.ds(l * 16, 16)] = jnp.maximum(
                    accs[LB + l] * (1.0 / S1), 0.0)
            return c
        lax.fori_loop(0, spc, sb, 0)

    epc = CH // S2
    nch = E2 // CH

    def pc_start(k, b):
        koff = pl.multiple_of(k * CH, CH)
        _issue_gather(pxn_hbm, ids2_v.at[pl.ds(koff, CH)], bufs[b], sems[b])

    def pc_compute(buf, k):
        def eb(e, c2):
            rowb = e * S2
            eg = k * epc + e
            srow = eg // S1
            _, vs = _load_pair(buf, rowb)
            for r in range(1, S2):
                _, vr = _load_pair(buf, rowb + r)
                vs = [a + b2 for a, b2 in zip(vs, vr)]
            for l in range(LB):
                v = jnp.maximum(vs[l] * (1.0 / S2), 0.0) * (1.0 / S1)
                plsc.addupdate(m1n_v.at[srow, pl.ds(l * 16, 16)], v)
            return c2
        lax.fori_loop(0, epc, eb, 0)

    def drain(b):
        _drain_gather(pxn_hbm.at[pl.ds(0, CH)], bufs[b], sems[b])

    dsub = rows_v.at[pl.ds(0, SPW)]

    _issue_gather(pxn_hbm, ids0_v, dsub, sems[0])
    _issue_gather(pxn_hbm, ids1_v.at[pl.ds(0, CH)], bufs[1], sems[1])
    _drain_gather(pxn_hbm.at[pl.ds(0, SPW)], dsub, sems[0])

    def pd(r, c):
        pxs, _ = _load_pair(rows_v, r)
        for l in range(LB):
            h0x_v[r, pl.ds(l * 16, 16)] = jnp.maximum(pxs[l], 0.0)
        return c
    lax.fori_loop(0, SPW, pd, 0)

    _issue_gather(pxn_hbm, ids1_v.at[pl.ds(CH, CH)], bufs[0], sems[0])
    drain(1)
    seed_mean2(bufs[1], 0)
    pc_start(0, 1)
    drain(0)
    seed_mean2(bufs[0], 1)
    pc_start(1, 0)

    def pc_pair(kp, c):
        k0 = pl.multiple_of(kp * 2, 2)
        drain(1)
        pc_compute(bufs[1], k0)

        @pl.when(kp < nch // 2 - 1)
        def _():
            pc_start(k0 + 2, 1)
        drain(0)
        pc_compute(bufs[0], k0 + 1)

        @pl.when(kp < nch // 2 - 1)
        def _():
            pc_start(k0 + 3, 0)
        return c
    lax.fori_loop(0, nch // 2, pc_pair, 0)

    pltpu.sync_copy(h0x_v, h0x_hbm.at[pl.ds(seed_base, SPW)])
    pltpu.sync_copy(h0n_v, h0n_hbm.at[pl.ds(seed_base, SPW)])
    pltpu.sync_copy(m1x_v, m1x_hbm.at[pl.ds(seed_base, SPW)])
    pltpu.sync_copy(m1n_v, m1n_hbm.at[pl.ds(seed_base, SPW)])


def _sc_gather(ids, ids1, ids2, pxn):
    mesh = plsc.VectorSubcoreMesh(core_axis_name="c", subcore_axis_name="s")
    f32, i32 = jnp.float32, jnp.int32
    out = jax.ShapeDtypeStruct((B, H), f32)
    k = pl.kernel(
        _sc_body,
        out_type=[out, out, out, out],
        mesh=mesh,
        compiler_params=pltpu.CompilerParams(needs_layout_passes=False),
        scratch_types=[
            pltpu.VMEM((SPW,), i32),
            pltpu.VMEM((E1,), i32),
            pltpu.VMEM((E2,), i32),
            pltpu.VMEM((CH, H), jnp.uint32),
            pltpu.VMEM((CH, H), jnp.uint32),
            pltpu.VMEM((SPW, H), f32),
            pltpu.VMEM((SPW, H), f32),
            pltpu.VMEM((SPW, H), f32),
            pltpu.VMEM((SPW, H), f32),
            pltpu.SemaphoreType.DMA,
            pltpu.SemaphoreType.DMA,
        ],
    )
    return k(ids, ids1, ids2, pxn)


def _head_body(h0x, h0n, m1x, m1n, wx2, wn2, wfc, bx2, bn2, bfc, out):
    gx = (jnp.dot(h0x[...], wx2[:H, :], preferred_element_type=jnp.float32)
          + jnp.dot(h0n[...], wx2[H:, :], preferred_element_type=jnp.float32)
          + bx2[...])
    gn = (jnp.dot(m1x[...], wn2[:H, :], preferred_element_type=jnp.float32)
          + jnp.dot(m1n[...], wn2[H:, :], preferred_element_type=jnp.float32)
          + bn2[...])
    nrm = (jnp.sum(gx * gx, axis=1, keepdims=True)
           + jnp.sum(gn * gn, axis=1, keepdims=True))
    s = 1.0 / jnp.maximum(jnp.sqrt(nrm), 1e-12)
    out[...] = (jnp.dot(gx * s, wfc[:H, :], preferred_element_type=jnp.float32)
                + jnp.dot(gn * s, wfc[H:, :],
                          preferred_element_type=jnp.float32)
                + bfc[...])


def _head(h0x, h0n, m1x, m1n, wx2, wn2, wfc, bx2, bn2, bfc):
    return pl.pallas_call(
        _head_body,
        out_shape=jax.ShapeDtypeStruct((B, N_CLASSES), jnp.float32),
    )(h0x, h0n, m1x, m1n, wx2, wn2, wfc, bx2, bn2, bfc)


def kernel(ids, feats, adj, perm1, perm2, W_x1, b_x1, W_n1, b_n1,
           W_x2, b_x2, W_n2, b_n2, W_fc, b_fc):
    wcat = jnp.concatenate([W_x1, W_n1], axis=1).astype(jnp.bfloat16)
    bcat = jnp.concatenate([b_x1, b_n1]).reshape(1, 2 * H)
    pxn = _project(feats, wcat, bcat)
    p1 = jnp.pad(perm1, (0, 32 - S1))
    p2 = jnp.pad(perm2, (0, 16 - S2))
    ids1, ids2 = _sc_ids(ids, adj.reshape(N_NODES // 4, 4 * MAX_DEG), p1, p2)
    h0x, h0n, m1x, m1n = _sc_gather(ids, ids1, ids2, pxn)
    return _head(h0x, h0n, m1x, m1n, W_x2, W_n2, W_fc,
                 b_x2.reshape(1, H), b_n2.reshape(1, H),
                 b_fc.reshape(1, N_CLASSES))


# speedup vs baseline: 4.8770x; 1.0003x over previous
import jax
import jax.numpy as jnp
from jax import lax
from jax.experimental import pallas as pl
from jax.experimental.pallas import tpu as pltpu
from jax.experimental.pallas import tpu_sc as plsc

N_NODES = 100000
MAX_DEG = 32
D_IN = 256
N_CLASSES = 40
B = 1024
S1, S2 = 25, 10
H = 128

NC, NS = 2, 16
NW = NC * NS
SPW = B // NW
E1 = SPW * S1
E2 = E1 * S2
CH = 400
CHA = 200
LB = H // 16


def _bf16_bits(x):
    u = jax.lax.bitcast_convert_type(x, jnp.uint32)
    return (u + jnp.uint32(0x7FFF) + ((u >> 16) & jnp.uint32(1))) >> 16


def _proj_body(f_ref, w_ref, b_ref, pxn_ref):
    acc = jnp.dot(f_ref[...].astype(jnp.bfloat16), w_ref[...],
                  preferred_element_type=jnp.float32)
    acc = acc + b_ref[...]
    px = _bf16_bits(acc[:, :H])
    pn = _bf16_bits(acc[:, H:])
    pxn_ref[...] = px | (pn << 16)


def _project(feats, wcat, bcat):
    rows = 10000
    grid = N_NODES // rows
    return pl.pallas_call(
        _proj_body,
        grid=(grid,),
        in_specs=[
            pl.BlockSpec((rows, D_IN), lambda i: (i, 0)),
            pl.BlockSpec((D_IN, 2 * H), lambda i: (0, 0)),
            pl.BlockSpec((1, 2 * H), lambda i: (0, 0)),
        ],
        out_specs=pl.BlockSpec((rows, H), lambda i: (i, 0)),
        out_shape=jax.ShapeDtypeStruct((N_NODES, H), jnp.uint32),
    )(feats, wcat, bcat)


def _gather_rows(table_hbm, idx_ref, dst_ref, sem):
    pltpu.async_copy(table_hbm.at[idx_ref], dst_ref, sem).wait()


def _issue_gather(table_hbm, idx_ref, dst_ref, sem):
    pltpu.async_copy(table_hbm.at[idx_ref], dst_ref, sem)


def _drain_gather(dummy_hbm, dst_ref, sem):
    pltpu.make_async_copy(dummy_hbm, dst_ref, sem).wait()


def _worker_id():
    return lax.axis_index("s") * NC + lax.axis_index("c")


def _load_pair(buf, row):
    pxs, pns = [], []
    for blk in range(LB):
        w = buf[row, pl.ds(blk * 16, 16)]
        pxs.append(plsc.bitcast(w << 16, jnp.float32))
        pns.append(plsc.bitcast(w & jnp.uint32(0xFFFF0000), jnp.float32))
    return pxs, pns


def _ids_body(ids_hbm, adjq_hbm, p1_hbm, p2_hbm,
              ids1_hbm, ids2_hbm,
              ids0_v, ids0q_v, p1_v, p2_v, adj0_v, ids1_v, ids1q_v, adjc_v,
              ids2_v, sem):
    wid = _worker_id()
    seed_base = wid * SPW

    pltpu.sync_copy(p1_hbm, p1_v)
    pltpu.sync_copy(p2_hbm, p2_v)
    pltpu.sync_copy(ids_hbm.at[pl.ds(seed_base, SPW)], ids0_v)

    def q0(i, c):
        ids0q_v[pl.ds(i * 16, 16)] = ids0_v[pl.ds(i * 16, 16)] // 4
        return c
    lax.fori_loop(0, SPW // 16, q0, 0)

    _gather_rows(adjq_hbm, ids0q_v, adj0_v, sem)

    def b1(i, c):
        mv = i * 16 + lax.iota(jnp.int32, 16)
        e = mv // S1
        nid = plsc.load_gather(ids0_v, [e])
        col = (nid % 4) * MAX_DEG + plsc.load_gather(p1_v, [mv % S1])
        val = plsc.load_gather(adj0_v, [e, col])
        ids1_v[pl.ds(i * 16, 16)] = val
        ids1q_v[pl.ds(i * 16, 16)] = val // 4
        return c
    lax.fori_loop(0, E1 // 16, b1, 0)

    for kc in range(E1 // CHA):
        _gather_rows(adjq_hbm, ids1q_v.at[pl.ds(kc * CHA, CHA)], adjc_v, sem)

        def b2(i, c):
            mg = kc * CHA * S2 + i * 16
            mv = mg + lax.iota(jnp.int32, 16)
            ent = mv // S2
            nid = plsc.load_gather(ids1_v, [ent])
            col = (nid % 4) * MAX_DEG + plsc.load_gather(p2_v, [mv % S2])
            ids2_v[pl.ds(mg, 16)] = plsc.load_gather(
                adjc_v, [ent - kc * CHA, col])
            return c
        lax.fori_loop(0, CHA * S2 // 16, b2, 0)

    pltpu.sync_copy(ids1_v, ids1_hbm.at[pl.ds(wid * E1, E1)])
    pltpu.sync_copy(ids2_v, ids2_hbm.at[pl.ds(wid * E2, E2)])


def _sc_ids(ids, adjq, p1, p2):
    mesh = plsc.VectorSubcoreMesh(core_axis_name="c", subcore_axis_name="s")
    i32 = jnp.int32
    k = pl.kernel(
        _ids_body,
        out_type=[jax.ShapeDtypeStruct((B * S1,), i32),
                  jax.ShapeDtypeStruct((B * S1 * S2,), i32)],
        mesh=mesh,
        compiler_params=pltpu.CompilerParams(needs_layout_passes=False),
        scratch_types=[
            pltpu.VMEM((SPW,), i32),
            pltpu.VMEM((SPW,), i32),
            pltpu.VMEM((32,), i32),
            pltpu.VMEM((16,), i32),
            pltpu.VMEM((SPW, 128), i32),
            pltpu.VMEM((E1,), i32),
            pltpu.VMEM((E1,), i32),
            pltpu.VMEM((CHA, 128), i32),
            pltpu.VMEM((E2,), i32),
            pltpu.SemaphoreType.DMA,
        ],
    )
    return k(ids, adjq, p1, p2)


def _sc_body(ids_hbm, ids1_hbm, ids2_hbm, pxn_hbm,
             h0x_hbm, h0n_hbm, m1x_hbm, m1n_hbm,
             ids0_v, ids1_v, ids2_v,
             rows_v, rows2_v, h0x_v, h0n_v, m1x_v, m1n_v, sem, sem2):
    wid = _worker_id()
    seed_base = wid * SPW

    pltpu.sync_copy(ids_hbm.at[pl.ds(seed_base, SPW)], ids0_v)
    pltpu.sync_copy(ids1_hbm.at[pl.ds(wid * E1, E1)], ids1_v)
    pltpu.sync_copy(ids2_hbm.at[pl.ds(wid * E2, E2)], ids2_v)

    bufs = (rows_v, rows2_v)
    sems = (sem, sem2)

    def seed_mean2(buf, k):
        spc = CH // S1

        def sb(s, c):
            def eb(e, accs):
                row = s * S1 + e
                pxs, pns = _load_pair(buf, row)
                ax = tuple(a + jnp.maximum(v, 0.0)
                           for a, v in zip(accs[:LB], pxs))
                an = tuple(a + v for a, v in zip(accs[LB:], pns))
                return ax + an
            accs = lax.fori_loop(
                0, S1, eb, tuple(jnp.zeros((16,), jnp.float32)
                                 for _ in range(2 * LB)))
            srow = k * spc + s
            for l in range(LB):
                m1x_v[srow, pl.ds(l * 16, 16)] = accs[l] * (1.0 / S1)
                h0n_v[srow, pl.ds(l * 16, 16)] = jnp.maximum(
                    accs[LB + l] * (1.0 / S1), 0.0)
            return c
        lax.fori_loop(0, spc, sb, 0)

    epc = CH // S2
    nch = E2 // CH

    def pc_start(k, b):
        koff = pl.multiple_of(k * CH, CH)
        _issue_gather(pxn_hbm, ids2_v.at[pl.ds(koff, CH)], bufs[b], sems[b])

    def pc_compute(buf, k, accs):
        def eb(e, accs):
            rowb = e * S2
            eg = k * epc + e
            srow = eg // S1
            _, vs = _load_pair(buf, rowb)
            for r in range(1, S2):
                _, vr = _load_pair(buf, rowb + r)
                vs = [a + b2 for a, b2 in zip(vs, vr)]
            accs = tuple(
                a + jnp.maximum(v * (1.0 / S2), 0.0) * (1.0 / S1)
                for a, v in zip(accs, vs))
            is_last = (eg % S1) == (S1 - 1)

            @pl.when(is_last)
            def _():
                for l in range(LB):
                    m1n_v[srow, pl.ds(l * 16, 16)] = accs[l]
            return tuple(
                jnp.where(is_last, jnp.zeros((16,), jnp.float32), a)
                for a in accs)
        return lax.fori_loop(0, epc, eb, accs)

    def drain(b):
        _drain_gather(pxn_hbm.at[pl.ds(0, CH)], bufs[b], sems[b])

    dsub = rows_v.at[pl.ds(0, SPW)]

    _issue_gather(pxn_hbm, ids0_v, dsub, sems[0])
    _issue_gather(pxn_hbm, ids1_v.at[pl.ds(0, CH)], bufs[1], sems[1])
    _drain_gather(pxn_hbm.at[pl.ds(0, SPW)], dsub, sems[0])

    def pd(r, c):
        pxs, _ = _load_pair(rows_v, r)
        for l in range(LB):
            h0x_v[r, pl.ds(l * 16, 16)] = jnp.maximum(pxs[l], 0.0)
        return c
    lax.fori_loop(0, SPW, pd, 0)

    _issue_gather(pxn_hbm, ids1_v.at[pl.ds(CH, CH)], bufs[0], sems[0])
    drain(1)
    seed_mean2(bufs[1], 0)
    pc_start(0, 1)
    drain(0)
    seed_mean2(bufs[0], 1)
    pc_start(1, 0)

    def pc_pair(kp, accs):
        k0 = pl.multiple_of(kp * 2, 2)
        drain(1)
        accs = pc_compute(bufs[1], k0, accs)

        @pl.when(kp < nch // 2 - 1)
        def _():
            pc_start(k0 + 2, 1)
        drain(0)
        accs = pc_compute(bufs[0], k0 + 1, accs)

        @pl.when(kp < nch // 2 - 1)
        def _():
            pc_start(k0 + 3, 0)
        return accs
    lax.fori_loop(0, nch // 2, pc_pair,
                  tuple(jnp.zeros((16,), jnp.float32) for _ in range(LB)))

    pltpu.sync_copy(h0x_v, h0x_hbm.at[pl.ds(seed_base, SPW)])
    pltpu.sync_copy(h0n_v, h0n_hbm.at[pl.ds(seed_base, SPW)])
    pltpu.sync_copy(m1x_v, m1x_hbm.at[pl.ds(seed_base, SPW)])
    pltpu.sync_copy(m1n_v, m1n_hbm.at[pl.ds(seed_base, SPW)])


def _sc_gather(ids, ids1, ids2, pxn):
    mesh = plsc.VectorSubcoreMesh(core_axis_name="c", subcore_axis_name="s")
    f32, i32 = jnp.float32, jnp.int32
    out = jax.ShapeDtypeStruct((B, H), f32)
    k = pl.kernel(
        _sc_body,
        out_type=[out, out, out, out],
        mesh=mesh,
        compiler_params=pltpu.CompilerParams(needs_layout_passes=False),
        scratch_types=[
            pltpu.VMEM((SPW,), i32),
            pltpu.VMEM((E1,), i32),
            pltpu.VMEM((E2,), i32),
            pltpu.VMEM((CH, H), jnp.uint32),
            pltpu.VMEM((CH, H), jnp.uint32),
            pltpu.VMEM((SPW, H), f32),
            pltpu.VMEM((SPW, H), f32),
            pltpu.VMEM((SPW, H), f32),
            pltpu.VMEM((SPW, H), f32),
            pltpu.SemaphoreType.DMA,
            pltpu.SemaphoreType.DMA,
        ],
    )
    return k(ids, ids1, ids2, pxn)


def _head_body(h0x, h0n, m1x, m1n, wx2, wn2, wfc, bx2, bn2, bfc, out):
    gx = (jnp.dot(h0x[...], wx2[:H, :], preferred_element_type=jnp.float32)
          + jnp.dot(h0n[...], wx2[H:, :], preferred_element_type=jnp.float32)
          + bx2[...])
    gn = (jnp.dot(m1x[...], wn2[:H, :], preferred_element_type=jnp.float32)
          + jnp.dot(m1n[...], wn2[H:, :], preferred_element_type=jnp.float32)
          + bn2[...])
    nrm = (jnp.sum(gx * gx, axis=1, keepdims=True)
           + jnp.sum(gn * gn, axis=1, keepdims=True))
    s = 1.0 / jnp.maximum(jnp.sqrt(nrm), 1e-12)
    out[...] = (jnp.dot(gx * s, wfc[:H, :], preferred_element_type=jnp.float32)
                + jnp.dot(gn * s, wfc[H:, :],
                          preferred_element_type=jnp.float32)
                + bfc[...])


def _head(h0x, h0n, m1x, m1n, wx2, wn2, wfc, bx2, bn2, bfc):
    return pl.pallas_call(
        _head_body,
        out_shape=jax.ShapeDtypeStruct((B, N_CLASSES), jnp.float32),
    )(h0x, h0n, m1x, m1n, wx2, wn2, wfc, bx2, bn2, bfc)


def kernel(ids, feats, adj, perm1, perm2, W_x1, b_x1, W_n1, b_n1,
           W_x2, b_x2, W_n2, b_n2, W_fc, b_fc):
    wcat = jnp.concatenate([W_x1, W_n1], axis=1).astype(jnp.bfloat16)
    bcat = jnp.concatenate([b_x1, b_n1]).reshape(1, 2 * H)
    pxn = _project(feats, wcat, bcat)
    p1 = jnp.pad(perm1, (0, 32 - S1))
    p2 = jnp.pad(perm2, (0, 16 - S2))
    ids1, ids2 = _sc_ids(ids, adj.reshape(N_NODES // 4, 4 * MAX_DEG), p1, p2)
    h0x, h0n, m1x, m1n = _sc_gather(ids, ids1, ids2, pxn)
    return _head(h0x, h0n, m1x, m1n, W_x2, W_n2, W_fc,
                 b_x2.reshape(1, H), b_n2.reshape(1, H),
                 b_fc.reshape(1, N_CLASSES))
